# Initial kernel scaffold; baseline (speedup 1.0000x reference)
import jax
import jax.numpy as jnp
from jax.experimental import pallas as pl


def kernel(atom_feature, coords, edge_index, edge_attr, params):
    raise NotImplementedError("write your pallas kernel here")


# speedup vs baseline: 1.3193x; 1.3193x over previous
import functools

import jax
import jax.numpy as jnp
from jax.experimental import pallas as pl

N = 10000
E = 320000
H = 128
ED = 16
XW = 16
TW = H + XW

BN = 2000
BE = 2000


def _silu(x):
    return x * jax.nn.sigmoid(x)


def _full(shape):
    return pl.BlockSpec(shape, lambda i: (0,) * len(shape))


def _rows(bsz, width):
    return pl.BlockSpec((bsz, width), lambda i: (i, 0))


def _pre_body(h0, w_in, b_in, w1a, w1b, b1, xp, h_out, ta_out, tb_out):
    h = jnp.dot(h0[...], w_in[...], preferred_element_type=jnp.float32) + b_in[...]
    pa = jnp.dot(h, w1a[...], preferred_element_type=jnp.float32) + b1[...]
    pb = jnp.dot(h, w1b[...], preferred_element_type=jnp.float32)
    h_out[...] = h
    ta_out[...] = jnp.concatenate([pa, xp[...]], axis=1)
    tb_out[...] = jnp.concatenate([pb, xp[...]], axis=1)


def _node_pre(h0, w_in, b_in, w1a, w1b, b1, xp):
    f32 = jnp.float32
    return pl.pallas_call(
        _pre_body,
        grid=(N // BN,),
        in_specs=[_rows(BN, H), _full((H, H)), _full((1, H)), _full((H, H)),
                  _full((H, H)), _full((1, H)), _rows(BN, XW)],
        out_specs=[_rows(BN, H), _rows(BN, TW), _rows(BN, TW)],
        out_shape=[jax.ShapeDtypeStruct((N, H), f32),
                   jax.ShapeDtypeStruct((N, TW), f32),
                   jax.ShapeDtypeStruct((N, TW), f32)],
    )(h0, w_in, b_in, w1a, w1b, b1, xp)


def _edge1_body(ga, gb, ea, w1e, rvec, b1r, w2, b2, cw1, cb1, c2w,
                m_out, aux_out):
    a = ga[...]
    b = gb[...]
    diff = a[:, H:] - b[:, H:]
    radial = jnp.sum(diff * diff, axis=1, keepdims=True)
    pre = (a[:, :H] + b[:, :H]
           + jnp.dot(ea[...], w1e[...], preferred_element_type=jnp.float32)
           + radial * rvec[...] + b1r[...])
    m1 = _silu(pre)
    m = _silu(jnp.dot(m1, w2[...], preferred_element_type=jnp.float32) + b2[...])
    cm = _silu(jnp.dot(m, cw1[...], preferred_element_type=jnp.float32) + cb1[...])
    s = jnp.sum(cm * c2w[...], axis=1, keepdims=True)
    lane = jax.lax.broadcasted_iota(jnp.int32, diff.shape, 1)
    aux = jnp.where(lane == 3, 1.0, diff * s)
    m_out[...] = m
    aux_out[...] = aux


def _edge1(ga, gb, ea, w1e, rvec, b1r, w2, b2, cw1, cb1, c2w):
    f32 = jnp.float32
    return pl.pallas_call(
        _edge1_body,
        grid=(E // BE,),
        in_specs=[_rows(BE, TW), _rows(BE, TW), _rows(BE, ED), _full((ED, H)),
                  _full((1, H)), _full((1, H)), _full((H, H)), _full((1, H)),
                  _full((H, H)), _full((1, H)), _full((1, H))],
        out_specs=[_rows(BE, H), _rows(BE, XW)],
        out_shape=[jax.ShapeDtypeStruct((E, H), f32),
                   jax.ShapeDtypeStruct((E, XW), f32)],
    )(ga, gb, ea, w1e, rvec, b1r, w2, b2, cw1, cb1, c2w)


def _edge2_body(ga, gb, ea, w1e, rvec, b1r, w2, b2, m_out):
    a = ga[...]
    b = gb[...]
    diff = a[:, H:] - b[:, H:]
    radial = jnp.sum(diff * diff, axis=1, keepdims=True)
    pre = (a[:, :H] + b[:, :H]
           + jnp.dot(ea[...], w1e[...], preferred_element_type=jnp.float32)
           + radial * rvec[...] + b1r[...])
    m1 = _silu(pre)
    m_out[...] = _silu(jnp.dot(m1, w2[...], preferred_element_type=jnp.float32)
                       + b2[...])


def _edge2(ga, gb, ea, w1e, rvec, b1r, w2, b2):
    return pl.pallas_call(
        _edge2_body,
        grid=(E // BE,),
        in_specs=[_rows(BE, TW), _rows(BE, TW), _rows(BE, ED), _full((ED, H)),
                  _full((1, H)), _full((1, H)), _full((H, H)), _full((1, H))],
        out_specs=_rows(BE, H),
        out_shape=jax.ShapeDtypeStruct((E, H), jnp.float32),
    )(ga, gb, ea, w1e, rvec, b1r, w2, b2)


def _node1_body(h, magg, aux, xp, nw1a, nw1b, nb1, nw2, nb2,
                w1a, w1b, b1, h_out, ta_out, tb_out):
    hh = h[...]
    av = aux[...]
    cnt = jnp.clip(av[:, 3:4], 1.0, None)
    lane = jax.lax.broadcasted_iota(jnp.int32, av.shape, 1)
    xp_new = xp[...] + jnp.where(lane < 3, av / cnt, 0.0)
    nh = _silu(jnp.dot(hh, nw1a[...], preferred_element_type=jnp.float32)
               + jnp.dot(magg[...], nw1b[...], preferred_element_type=jnp.float32)
               + nb1[...])
    h_new = hh + jnp.dot(nh, nw2[...], preferred_element_type=jnp.float32) + nb2[...]
    pa = jnp.dot(h_new, w1a[...], preferred_element_type=jnp.float32) + b1[...]
    pb = jnp.dot(h_new, w1b[...], preferred_element_type=jnp.float32)
    h_out[...] = h_new
    ta_out[...] = jnp.concatenate([pa, xp_new], axis=1)
    tb_out[...] = jnp.concatenate([pb, xp_new], axis=1)


def _node1(h, magg, aux, xp, nw1a, nw1b, nb1, nw2, nb2, w1a, w1b, b1):
    f32 = jnp.float32
    return pl.pallas_call(
        _node1_body,
        grid=(N // BN,),
        in_specs=[_rows(BN, H), _rows(BN, H), _rows(BN, XW), _rows(BN, XW),
                  _full((H, H)), _full((H, H)), _full((1, H)), _full((H, H)),
                  _full((1, H)), _full((H, H)), _full((H, H)), _full((1, H))],
        out_specs=[_rows(BN, H), _rows(BN, TW), _rows(BN, TW)],
        out_shape=[jax.ShapeDtypeStruct((N, H), f32),
                   jax.ShapeDtypeStruct((N, TW), f32),
                   jax.ShapeDtypeStruct((N, TW), f32)],
    )(h, magg, aux, xp, nw1a, nw1b, nb1, nw2, nb2, w1a, w1b, b1)


def _nodeout_body(h, magg, nw1a, nw1b, nb1, nw2, nb2, wo, bo, out):
    hh = h[...]
    nh = _silu(jnp.dot(hh, nw1a[...], preferred_element_type=jnp.float32)
               + jnp.dot(magg[...], nw1b[...], preferred_element_type=jnp.float32)
               + nb1[...])
    h_new = hh + jnp.dot(nh, nw2[...], preferred_element_type=jnp.float32) + nb2[...]
    out[...] = jnp.dot(h_new, wo[...], preferred_element_type=jnp.float32) + bo[...]


def _node_out(h, magg, nw1a, nw1b, nb1, nw2, nb2, wo, bo):
    return pl.pallas_call(
        _nodeout_body,
        grid=(N // BN,),
        in_specs=[_rows(BN, H), _rows(BN, H), _full((H, H)), _full((H, H)),
                  _full((1, H)), _full((H, H)), _full((1, H)), _full((H, H)),
                  _full((1, H))],
        out_specs=_rows(BN, H),
        out_shape=jax.ShapeDtypeStruct((N, H), jnp.float32),
    )(h, magg, nw1a, nw1b, nb1, nw2, nb2, wo, bo)


def kernel(atom_feature, coords, edge_index, edge_attr, params):
    f32 = jnp.float32
    row = edge_index[0].astype(jnp.int32)
    col = edge_index[1].astype(jnp.int32)
    xp0 = jnp.pad(coords.astype(f32), ((0, 0), (0, XW - 3)))

    w_in, b_in = params['emb_in']
    wo, bo = params['emb_out']
    lps = params['layers']

    def split_l(lp):
        w1, b1 = lp['edge1']
        return dict(
            w1a=w1[:H], w1b=w1[H:2 * H], rvec=w1[2 * H:2 * H + 1],
            w1e=w1[2 * H + 1:], b1=b1.reshape(1, H),
            w2=lp['edge2'][0], b2=lp['edge2'][1].reshape(1, H),
            cw1=lp['coord1'][0], cb1=lp['coord1'][1].reshape(1, H),
            c2w=lp['coord2_w'].reshape(1, H),
            nw1a=lp['node1'][0][:H], nw1b=lp['node1'][0][H:],
            nb1=lp['node1'][1].reshape(1, H),
            nw2=lp['node2'][0], nb2=lp['node2'][1].reshape(1, H),
        )

    l1, l2 = split_l(lps[0]), split_l(lps[1])

    h, ta, tb = _node_pre(atom_feature, w_in, b_in.reshape(1, H),
                          l1['w1a'], l1['w1b'], jnp.zeros((1, H), f32), xp0)
    ga = jnp.take(ta, row, axis=0)
    gb = jnp.take(tb, col, axis=0)
    m, aux = _edge1(ga, gb, edge_attr, l1['w1e'], l1['rvec'], l1['b1'],
                    l1['w2'], l1['b2'], l1['cw1'], l1['cb1'], l1['c2w'])
    magg = jax.ops.segment_sum(m, row, num_segments=N)
    auxagg = jax.ops.segment_sum(aux, row, num_segments=N)
    h2, ta2, tb2 = _node1(h, magg, auxagg, xp0, l1['nw1a'], l1['nw1b'],
                          l1['nb1'], l1['nw2'], l1['nb2'],
                          l2['w1a'], l2['w1b'], jnp.zeros((1, H), f32))

    ga2 = jnp.take(ta2, row, axis=0)
    gb2 = jnp.take(tb2, col, axis=0)
    m2 = _edge2(ga2, gb2, edge_attr, l2['w1e'], l2['rvec'], l2['b1'],
                l2['w2'], l2['b2'])
    magg2 = jax.ops.segment_sum(m2, row, num_segments=N)
    return _node_out(h2, magg2, l2['nw1a'], l2['nw1b'], l2['nb1'],
                     l2['nw2'], l2['nb2'], wo, bo.reshape(1, H))


# speedup vs baseline: 1.3770x; 1.0438x over previous
import functools

import jax
import jax.numpy as jnp
from jax import lax
from jax.experimental import pallas as pl
from jax.experimental.pallas import tpu as pltpu
from jax.experimental.pallas import tpu_sc as plsc

N = 10000
E = 320000
H = 128
ED = 16

BN = 2000
BE = 2000


def _silu(x):
    return x * jax.nn.sigmoid(x)


def _full(shape):
    return pl.BlockSpec(shape, lambda i: (0,) * len(shape))


def _rows(bsz, width):
    return pl.BlockSpec((bsz, width), lambda i: (i, 0))


def _pre_body(h0, w_in, b_in, w1a, w1b, h_out, ta_out, tb_out):
    h = jnp.dot(h0[...], w_in[...], preferred_element_type=jnp.float32) + b_in[...]
    h_out[...] = h
    ta_out[...] = jnp.dot(h, w1a[...], preferred_element_type=jnp.float32)
    tb_out[...] = jnp.dot(h, w1b[...], preferred_element_type=jnp.float32)


def _node_pre(h0, w_in, b_in, w1a, w1b):
    f32 = jnp.float32
    return pl.pallas_call(
        _pre_body,
        grid=(N // BN,),
        in_specs=[_rows(BN, H), _full((H, H)), _full((1, H)), _full((H, H)),
                  _full((H, H))],
        out_specs=[_rows(BN, H), _rows(BN, H), _rows(BN, H)],
        out_shape=[jax.ShapeDtypeStruct((N, H), f32),
                   jax.ShapeDtypeStruct((N, H), f32),
                   jax.ShapeDtypeStruct((N, H), f32)],
    )(h0, w_in, b_in, w1a, w1b)


def _edge1_body(ga, gb, xr, xc, ea, w1e, rvec, b1r, w2, b2, cw1, cb1, c2w,
                m_out, aux_out):
    diff = xr[...] - xc[...]
    radial = jnp.sum(diff * diff, axis=1, keepdims=True)
    pre = (ga[...] + gb[...]
           + jnp.dot(ea[...], w1e[...], preferred_element_type=jnp.float32)
           + radial * rvec[...] + b1r[...])
    m1 = _silu(pre)
    m = _silu(jnp.dot(m1, w2[...], preferred_element_type=jnp.float32) + b2[...])
    cm = _silu(jnp.dot(m, cw1[...], preferred_element_type=jnp.float32) + cb1[...])
    s = jnp.sum(cm * c2w[...], axis=1, keepdims=True)
    lane = jax.lax.broadcasted_iota(jnp.int32, diff.shape, 1)
    aux = jnp.where(lane == 3, 1.0, diff * s)
    m_out[...] = m
    aux_out[...] = aux


def _edge1(ga, gb, xr, xc, ea, w1e, rvec, b1r, w2, b2, cw1, cb1, c2w):
    f32 = jnp.float32
    return pl.pallas_call(
        _edge1_body,
        grid=(E // BE,),
        in_specs=[_rows(BE, H), _rows(BE, H), _rows(BE, 4), _rows(BE, 4),
                  _rows(BE, ED), _full((ED, H)), _full((1, H)), _full((1, H)),
                  _full((H, H)), _full((1, H)), _full((H, H)), _full((1, H)),
                  _full((1, H))],
        out_specs=[_rows(BE, H), _rows(BE, 4)],
        out_shape=[jax.ShapeDtypeStruct((E, H), f32),
                   jax.ShapeDtypeStruct((E, 4), f32)],
    )(ga, gb, xr, xc, ea, w1e, rvec, b1r, w2, b2, cw1, cb1, c2w)


def _edge2_body(ga, gb, xr, xc, ea, w1e, rvec, b1r, w2, b2, m_out):
    diff = xr[...] - xc[...]
    radial = jnp.sum(diff * diff, axis=1, keepdims=True)
    pre = (ga[...] + gb[...]
           + jnp.dot(ea[...], w1e[...], preferred_element_type=jnp.float32)
           + radial * rvec[...] + b1r[...])
    m1 = _silu(pre)
    m_out[...] = _silu(jnp.dot(m1, w2[...], preferred_element_type=jnp.float32)
                       + b2[...])


def _edge2(ga, gb, xr, xc, ea, w1e, rvec, b1r, w2, b2):
    return pl.pallas_call(
        _edge2_body,
        grid=(E // BE,),
        in_specs=[_rows(BE, H), _rows(BE, H), _rows(BE, 4), _rows(BE, 4),
                  _rows(BE, ED), _full((ED, H)), _full((1, H)), _full((1, H)),
                  _full((H, H)), _full((1, H))],
        out_specs=_rows(BE, H),
        out_shape=jax.ShapeDtypeStruct((E, H), jnp.float32),
    )(ga, gb, xr, xc, ea, w1e, rvec, b1r, w2, b2)


def _node1_body(h, magg, aux, x4, nw1a, nw1b, nb1, nw2, nb2,
                w1a, w1b, h_out, ta_out, tb_out, x_out):
    hh = h[...]
    av = aux[...]
    cnt = jnp.clip(av[:, 3:4], 1.0, None)
    lane = jax.lax.broadcasted_iota(jnp.int32, av.shape, 1)
    x_out[...] = x4[...] + jnp.where(lane < 3, av / cnt, 0.0)
    nh = _silu(jnp.dot(hh, nw1a[...], preferred_element_type=jnp.float32)
               + jnp.dot(magg[...], nw1b[...], preferred_element_type=jnp.float32)
               + nb1[...])
    h_new = hh + jnp.dot(nh, nw2[...], preferred_element_type=jnp.float32) + nb2[...]
    h_out[...] = h_new
    ta_out[...] = jnp.dot(h_new, w1a[...], preferred_element_type=jnp.float32)
    tb_out[...] = jnp.dot(h_new, w1b[...], preferred_element_type=jnp.float32)


def _node1(h, magg, aux, x4, nw1a, nw1b, nb1, nw2, nb2, w1a, w1b):
    f32 = jnp.float32
    return pl.pallas_call(
        _node1_body,
        grid=(N // BN,),
        in_specs=[_rows(BN, H), _rows(BN, H), _rows(BN, 4), _rows(BN, 4),
                  _full((H, H)), _full((H, H)), _full((1, H)), _full((H, H)),
                  _full((1, H)), _full((H, H)), _full((H, H))],
        out_specs=[_rows(BN, H), _rows(BN, H), _rows(BN, H), _rows(BN, 4)],
        out_shape=[jax.ShapeDtypeStruct((N, H), f32),
                   jax.ShapeDtypeStruct((N, H), f32),
                   jax.ShapeDtypeStruct((N, H), f32),
                   jax.ShapeDtypeStruct((N, 4), f32)],
    )(h, magg, aux, x4, nw1a, nw1b, nb1, nw2, nb2, w1a, w1b)


def _nodeout_body(h, magg, nw1a, nw1b, nb1, nw2, nb2, wo, bo, out):
    hh = h[...]
    nh = _silu(jnp.dot(hh, nw1a[...], preferred_element_type=jnp.float32)
               + jnp.dot(magg[...], nw1b[...], preferred_element_type=jnp.float32)
               + nb1[...])
    h_new = hh + jnp.dot(nh, nw2[...], preferred_element_type=jnp.float32) + nb2[...]
    out[...] = jnp.dot(h_new, wo[...], preferred_element_type=jnp.float32) + bo[...]


def _node_out(h, magg, nw1a, nw1b, nb1, nw2, nb2, wo, bo):
    return pl.pallas_call(
        _nodeout_body,
        grid=(N // BN,),
        in_specs=[_rows(BN, H), _rows(BN, H), _full((H, H)), _full((H, H)),
                  _full((1, H)), _full((H, H)), _full((1, H)), _full((H, H)),
                  _full((1, H))],
        out_specs=_rows(BN, H),
        out_shape=jax.ShapeDtypeStruct((N, H), jnp.float32),
    )(h, magg, nw1a, nw1b, nb1, nw2, nb2, wo, bo)


NS = 16
EPT = E // NS
CG = 80
NB = 5
CPT = EPT // CG
NG = CPT // NB


def _gather_body(ta_h, tb_h, r_h, c_h, ga_h, gb_h, idxv, buf, *sems):
    gsems, ssems = sems[:NB], sems[NB:]
    cid = lax.axis_index("c")
    sid = lax.axis_index("s")

    def run(tab, ih, oh):
        base = sid * EPT
        pltpu.sync_copy(ih.at[sid], idxv)

        def grp(g, _):
            handles = []
            for b in range(NB):
                j = g * NB + b

                @pl.when(g > 0)
                def _():
                    pltpu.make_async_copy(buf.at[b], oh.at[pl.ds(0, CG)],
                                          ssems[b]).wait()

                handles.append(pltpu.async_copy(tab.at[idxv.at[j]], buf.at[b],
                                                gsems[b]))
            for b in range(NB):
                off = base + (g * NB + b) * CG
                handles[b].wait()
                pltpu.async_copy(buf.at[b], oh.at[pl.ds(off, CG)], ssems[b])
            return ()

        lax.fori_loop(0, NG, grp, ())
        for b in range(NB):
            pltpu.make_async_copy(buf.at[b], oh.at[pl.ds(0, CG)],
                                  ssems[b]).wait()

    @pl.when(cid == 0)
    def _():
        run(ta_h, r_h, ga_h)

    @pl.when(cid == 1)
    def _():
        run(tb_h, c_h, gb_h)


def _sc_gather(ta, tb, row3d, col3d):
    f32 = jnp.float32
    fn = pl.kernel(
        _gather_body,
        out_type=[jax.ShapeDtypeStruct((E, H), f32),
                  jax.ShapeDtypeStruct((E, H), f32)],
        mesh=plsc.VectorSubcoreMesh(core_axis_name="c", subcore_axis_name="s"),
        scratch_types=[pltpu.VMEM((CPT, CG), jnp.int32),
                       pltpu.VMEM((NB, CG, H), f32)]
        + [pltpu.SemaphoreType.DMA] * (2 * NB),
    )
    return fn(ta, tb, row3d, col3d)


def kernel(atom_feature, coords, edge_index, edge_attr, params):
    f32 = jnp.float32
    row = edge_index[0].astype(jnp.int32)
    col = edge_index[1].astype(jnp.int32)
    x40 = jnp.pad(coords.astype(f32), ((0, 0), (0, 1)))

    w_in, b_in = params['emb_in']
    wo, bo = params['emb_out']
    lps = params['layers']

    def split_l(lp):
        w1, b1 = lp['edge1']
        return dict(
            w1a=w1[:H], w1b=w1[H:2 * H], rvec=w1[2 * H:2 * H + 1],
            w1e=w1[2 * H + 1:], b1=b1.reshape(1, H),
            w2=lp['edge2'][0], b2=lp['edge2'][1].reshape(1, H),
            cw1=lp['coord1'][0], cb1=lp['coord1'][1].reshape(1, H),
            c2w=lp['coord2_w'].reshape(1, H),
            nw1a=lp['node1'][0][:H], nw1b=lp['node1'][0][H:],
            nb1=lp['node1'][1].reshape(1, H),
            nw2=lp['node2'][0], nb2=lp['node2'][1].reshape(1, H),
        )

    l1, l2 = split_l(lps[0]), split_l(lps[1])
    row3d = row.reshape(NS, CPT, CG)
    col3d = col.reshape(NS, CPT, CG)

    h, ta, tb = _node_pre(atom_feature, w_in, b_in.reshape(1, H),
                          l1['w1a'], l1['w1b'])
    ga, gb = _sc_gather(ta, tb, row3d, col3d)
    xr = jnp.take(x40, row, axis=0)
    xc = jnp.take(x40, col, axis=0)
    m, aux = _edge1(ga, gb, xr, xc, edge_attr, l1['w1e'], l1['rvec'], l1['b1'],
                    l1['w2'], l1['b2'], l1['cw1'], l1['cb1'], l1['c2w'])
    magg = jax.ops.segment_sum(m, row, num_segments=N)
    auxagg = jax.ops.segment_sum(aux, row, num_segments=N)
    h2, ta2, tb2, x41 = _node1(h, magg, auxagg, x40, l1['nw1a'], l1['nw1b'],
                               l1['nb1'], l1['nw2'], l1['nb2'],
                               l2['w1a'], l2['w1b'])

    ga2, gb2 = _sc_gather(ta2, tb2, row3d, col3d)
    xr2 = jnp.take(x41, row, axis=0)
    xc2 = jnp.take(x41, col, axis=0)
    m2 = _edge2(ga2, gb2, xr2, xc2, edge_attr, l2['w1e'], l2['rvec'], l2['b1'],
                l2['w2'], l2['b2'])
    magg2 = jax.ops.segment_sum(m2, row, num_segments=N)
    return _node_out(h2, magg2, l2['nw1a'], l2['nw1b'], l2['nb1'],
                     l2['nw2'], l2['nb2'], wo, bo.reshape(1, H))


# speedup vs baseline: 1.6521x; 1.1998x over previous
import functools

import jax
import jax.numpy as jnp
from jax import lax
from jax.experimental import pallas as pl
from jax.experimental.pallas import tpu as pltpu
from jax.experimental.pallas import tpu_sc as plsc

N = 10000
E = 320000
H = 128
ED = 16

BN = 2000
BE = 2000


def _silu(x):
    return x * jax.nn.sigmoid(x)


def _full(shape):
    return pl.BlockSpec(shape, lambda i: (0,) * len(shape))


def _rows(bsz, width):
    return pl.BlockSpec((bsz, width), lambda i: (i, 0))


def _pre_body(h0, w_in, b_in, w1a, w1b, h_out, ta_out, tb_out):
    h = jnp.dot(h0[...], w_in[...], preferred_element_type=jnp.float32) + b_in[...]
    h_out[...] = h
    ta_out[...] = jnp.dot(h, w1a[...], preferred_element_type=jnp.float32)
    tb_out[...] = jnp.dot(h, w1b[...], preferred_element_type=jnp.float32)


def _node_pre(h0, w_in, b_in, w1a, w1b):
    f32 = jnp.float32
    return pl.pallas_call(
        _pre_body,
        grid=(N // BN,),
        in_specs=[_rows(BN, H), _full((H, H)), _full((1, H)), _full((H, H)),
                  _full((H, H))],
        out_specs=[_rows(BN, H), _rows(BN, H), _rows(BN, H)],
        out_shape=[jax.ShapeDtypeStruct((N, H), f32),
                   jax.ShapeDtypeStruct((N, H), f32),
                   jax.ShapeDtypeStruct((N, H), f32)],
    )(h0, w_in, b_in, w1a, w1b)


def _edge1_body(ga, gb, xr, xc, ea, w1e, rvec, b1r, w2, b2, cw1, cb1, c2w,
                m_out, aux_out):
    diff = xr[...] - xc[...]
    radial = jnp.sum(diff * diff, axis=1, keepdims=True)
    pre = (ga[...] + gb[...]
           + jnp.dot(ea[...], w1e[...], preferred_element_type=jnp.float32)
           + radial * rvec[...] + b1r[...])
    m1 = _silu(pre)
    m = _silu(jnp.dot(m1, w2[...], preferred_element_type=jnp.float32) + b2[...])
    cm = _silu(jnp.dot(m, cw1[...], preferred_element_type=jnp.float32) + cb1[...])
    s = jnp.sum(cm * c2w[...], axis=1, keepdims=True)
    lane = jax.lax.broadcasted_iota(jnp.int32, diff.shape, 1)
    aux = jnp.where(lane == 3, 1.0, diff * s)
    m_out[...] = m
    aux_out[...] = aux


def _edge1(ga, gb, xr, xc, ea, w1e, rvec, b1r, w2, b2, cw1, cb1, c2w):
    f32 = jnp.float32
    return pl.pallas_call(
        _edge1_body,
        grid=(E // BE,),
        in_specs=[_rows(BE, H), _rows(BE, H), _rows(BE, 4), _rows(BE, 4),
                  _rows(BE, ED), _full((ED, H)), _full((1, H)), _full((1, H)),
                  _full((H, H)), _full((1, H)), _full((H, H)), _full((1, H)),
                  _full((1, H))],
        out_specs=[_rows(BE, H), _rows(BE, 4)],
        out_shape=[jax.ShapeDtypeStruct((E, H), f32),
                   jax.ShapeDtypeStruct((E, 4), f32)],
    )(ga, gb, xr, xc, ea, w1e, rvec, b1r, w2, b2, cw1, cb1, c2w)


def _edge2_body(ga, gb, xr, xc, ea, w1e, rvec, b1r, w2, b2, m_out):
    diff = xr[...] - xc[...]
    radial = jnp.sum(diff * diff, axis=1, keepdims=True)
    pre = (ga[...] + gb[...]
           + jnp.dot(ea[...], w1e[...], preferred_element_type=jnp.float32)
           + radial * rvec[...] + b1r[...])
    m1 = _silu(pre)
    m_out[...] = _silu(jnp.dot(m1, w2[...], preferred_element_type=jnp.float32)
                       + b2[...])


def _edge2(ga, gb, xr, xc, ea, w1e, rvec, b1r, w2, b2):
    return pl.pallas_call(
        _edge2_body,
        grid=(E // BE,),
        in_specs=[_rows(BE, H), _rows(BE, H), _rows(BE, 4), _rows(BE, 4),
                  _rows(BE, ED), _full((ED, H)), _full((1, H)), _full((1, H)),
                  _full((H, H)), _full((1, H))],
        out_specs=_rows(BE, H),
        out_shape=jax.ShapeDtypeStruct((E, H), jnp.float32),
    )(ga, gb, xr, xc, ea, w1e, rvec, b1r, w2, b2)


def _node1_body(h, mp0, mp1, aux, x4, nw1a, nw1b, nb1, nw2, nb2,
                w1a, w1b, h_out, ta_out, tb_out, x_out):
    hh = h[...]
    magg = mp0[0] + mp1[0]
    av = aux[...]
    cnt = jnp.clip(av[:, 3:4], 1.0, None)
    lane = jax.lax.broadcasted_iota(jnp.int32, av.shape, 1)
    x_out[...] = x4[...] + jnp.where(lane < 3, av / cnt, 0.0)
    nh = _silu(jnp.dot(hh, nw1a[...], preferred_element_type=jnp.float32)
               + jnp.dot(magg, nw1b[...], preferred_element_type=jnp.float32)
               + nb1[...])
    h_new = hh + jnp.dot(nh, nw2[...], preferred_element_type=jnp.float32) + nb2[...]
    h_out[...] = h_new
    ta_out[...] = jnp.dot(h_new, w1a[...], preferred_element_type=jnp.float32)
    tb_out[...] = jnp.dot(h_new, w1b[...], preferred_element_type=jnp.float32)


def _part(bsz, width, p):
    return pl.BlockSpec((1, bsz, width), lambda i, _p=p: (_p, i, 0))


def _node1(h, mp, aux, x4, nw1a, nw1b, nb1, nw2, nb2, w1a, w1b):
    f32 = jnp.float32
    return pl.pallas_call(
        _node1_body,
        grid=(N // BN,),
        in_specs=[_rows(BN, H), _part(BN, H, 0), _part(BN, H, 1),
                  _rows(BN, 4), _rows(BN, 4),
                  _full((H, H)), _full((H, H)), _full((1, H)), _full((H, H)),
                  _full((1, H)), _full((H, H)), _full((H, H))],
        out_specs=[_rows(BN, H), _rows(BN, H), _rows(BN, H), _rows(BN, 4)],
        out_shape=[jax.ShapeDtypeStruct((N, H), f32),
                   jax.ShapeDtypeStruct((N, H), f32),
                   jax.ShapeDtypeStruct((N, H), f32),
                   jax.ShapeDtypeStruct((N, 4), f32)],
    )(h, mp, mp, aux, x4, nw1a, nw1b, nb1, nw2, nb2, w1a, w1b)


def _nodeout_body(h, mp0, mp1, nw1a, nw1b, nb1, nw2, nb2, wo, bo, out):
    hh = h[...]
    magg = mp0[0] + mp1[0]
    nh = _silu(jnp.dot(hh, nw1a[...], preferred_element_type=jnp.float32)
               + jnp.dot(magg, nw1b[...], preferred_element_type=jnp.float32)
               + nb1[...])
    h_new = hh + jnp.dot(nh, nw2[...], preferred_element_type=jnp.float32) + nb2[...]
    out[...] = jnp.dot(h_new, wo[...], preferred_element_type=jnp.float32) + bo[...]


def _node_out(h, mp, nw1a, nw1b, nb1, nw2, nb2, wo, bo):
    return pl.pallas_call(
        _nodeout_body,
        grid=(N // BN,),
        in_specs=[_rows(BN, H), _part(BN, H, 0), _part(BN, H, 1),
                  _full((H, H)), _full((H, H)),
                  _full((1, H)), _full((H, H)), _full((1, H)), _full((H, H)),
                  _full((1, H))],
        out_specs=_rows(BN, H),
        out_shape=jax.ShapeDtypeStruct((N, H), jnp.float32),
    )(h, mp, mp, nw1a, nw1b, nb1, nw2, nb2, wo, bo)


NS = 16
EPT = E // NS
CG = 80
NB = 5
CPT = EPT // CG
NG = CPT // NB


def _gather_body(ta_h, tb_h, r_h, c_h, ga_h, gb_h, idxv, buf, *sems):
    gsems, ssems = sems[:NB], sems[NB:]
    cid = lax.axis_index("c")
    sid = lax.axis_index("s")

    def run(tab, ih, oh):
        base = sid * EPT
        pltpu.sync_copy(ih.at[sid], idxv)

        def grp(g, _):
            handles = []
            for b in range(NB):
                j = g * NB + b

                @pl.when(g > 0)
                def _():
                    pltpu.make_async_copy(buf.at[b], oh.at[pl.ds(0, CG)],
                                          ssems[b]).wait()

                handles.append(pltpu.async_copy(tab.at[idxv.at[j]], buf.at[b],
                                                gsems[b]))
            for b in range(NB):
                off = base + (g * NB + b) * CG
                handles[b].wait()
                pltpu.async_copy(buf.at[b], oh.at[pl.ds(off, CG)], ssems[b])
            return ()

        lax.fori_loop(0, NG, grp, ())
        for b in range(NB):
            pltpu.make_async_copy(buf.at[b], oh.at[pl.ds(0, CG)],
                                  ssems[b]).wait()

    @pl.when(cid == 0)
    def _():
        run(ta_h, r_h, ga_h)

    @pl.when(cid == 1)
    def _():
        run(tb_h, c_h, gb_h)


def _sc_gather(ta, tb, row3d, col3d):
    f32 = jnp.float32
    fn = pl.kernel(
        _gather_body,
        out_type=[jax.ShapeDtypeStruct((E, H), f32),
                  jax.ShapeDtypeStruct((E, H), f32)],
        mesh=plsc.VectorSubcoreMesh(core_axis_name="c", subcore_axis_name="s"),
        scratch_types=[pltpu.VMEM((CPT, CG), jnp.int32),
                       pltpu.VMEM((NB, CG, H), f32)]
        + [pltpu.SemaphoreType.DMA] * (2 * NB),
    )
    return fn(ta, tb, row3d, col3d)


NP = 10240
RPT = NP // NS
CGS = 40
NBS = 5
CPTS = (E // 32) // CGS


NGS = CPTS // NBS


def _scatter_body(m_h, r_h, z_h, om_h, idxg, mring, accm, *sems):
    asems = sems[:NBS]
    zsem, isem = sems[NBS], sems[NBS + 1]
    cid = lax.axis_index("c")
    sid = lax.axis_index("s")

    pltpu.sync_copy(z_h, mring.at[0])
    for i in range(RPT // CGS):
        pltpu.async_copy(mring.at[0], accm.at[pl.ds(sid * RPT + i * CGS, CGS)],
                         zsem)
    for i in range(RPT // CGS):
        pltpu.make_async_copy(mring.at[0], accm.at[pl.ds(0, CGS)], zsem).wait()
    plsc.subcore_barrier()

    tid = cid * NS + sid
    base = tid * (E // 32)
    r_t = r_h.at[tid]
    pltpu.sync_copy(r_t.at[0], idxg.at[0])

    def grp(g, _):
        pb = lax.rem(g, 2)

        @pl.when(g > 0)
        def _():
            pltpu.make_async_copy(r_t.at[0], idxg.at[0], isem).wait()

        handles = []
        for b in range(NBS):
            off = base + (g * NBS + b) * CGS

            @pl.when(g > 0)
            def _():
                pltpu.make_async_copy(mring.at[b], accm.at[pl.ds(0, CGS)],
                                      asems[b]).wait()

            handles.append(pltpu.async_copy(m_h.at[pl.ds(off, CGS)],
                                            mring.at[b], asems[b]))
        for b in range(NBS):
            handles[b].wait()
            pltpu.async_copy(mring.at[b], accm.at[idxg.at[pb, b]],
                             asems[b], add=True)

        @pl.when(g + 1 < NGS)
        def _():
            pltpu.async_copy(r_t.at[g + 1], idxg.at[1 - pb], isem)

        return ()

    lax.fori_loop(0, NGS, grp, ())
    for b in range(NBS):
        pltpu.make_async_copy(mring.at[b], accm.at[pl.ds(0, CGS)],
                              asems[b]).wait()
    plsc.subcore_barrier()

    @pl.when(sid < NS - 1)
    def _():
        pltpu.sync_copy(accm.at[pl.ds(sid * RPT, RPT)],
                        om_h.at[cid].at[pl.ds(sid * RPT, RPT)])

    @pl.when(sid == NS - 1)
    def _():
        pltpu.sync_copy(accm.at[pl.ds((NS - 1) * RPT, 400)],
                        om_h.at[cid].at[pl.ds((NS - 1) * RPT, 400)])


def _sc_scatter(m, row4d, zer):
    f32 = jnp.float32
    fn = pl.kernel(
        _scatter_body,
        out_type=jax.ShapeDtypeStruct((2, N, H), f32),
        mesh=plsc.VectorSubcoreMesh(core_axis_name="c", subcore_axis_name="s"),
        scratch_types=[pltpu.VMEM((2, NBS, CGS), jnp.int32),
                       pltpu.VMEM((NBS, CGS, H), f32),
                       pltpu.VMEM_SHARED((NP, H), f32)]
        + [pltpu.SemaphoreType.DMA] * (NBS + 2),
    )
    return fn(m, row4d, zer)


def kernel(atom_feature, coords, edge_index, edge_attr, params):
    f32 = jnp.float32
    row = edge_index[0].astype(jnp.int32)
    col = edge_index[1].astype(jnp.int32)
    x40 = jnp.pad(coords.astype(f32), ((0, 0), (0, 1)))

    w_in, b_in = params['emb_in']
    wo, bo = params['emb_out']
    lps = params['layers']

    def split_l(lp):
        w1, b1 = lp['edge1']
        return dict(
            w1a=w1[:H], w1b=w1[H:2 * H], rvec=w1[2 * H:2 * H + 1],
            w1e=w1[2 * H + 1:], b1=b1.reshape(1, H),
            w2=lp['edge2'][0], b2=lp['edge2'][1].reshape(1, H),
            cw1=lp['coord1'][0], cb1=lp['coord1'][1].reshape(1, H),
            c2w=lp['coord2_w'].reshape(1, H),
            nw1a=lp['node1'][0][:H], nw1b=lp['node1'][0][H:],
            nb1=lp['node1'][1].reshape(1, H),
            nw2=lp['node2'][0], nb2=lp['node2'][1].reshape(1, H),
        )

    l1, l2 = split_l(lps[0]), split_l(lps[1])
    row3d = row.reshape(NS, CPT, CG)
    col3d = col.reshape(NS, CPT, CG)
    row4d = row.reshape(32, NGS, NBS, CGS)
    zer = jnp.zeros((CGS, H), f32)

    h, ta, tb = _node_pre(atom_feature, w_in, b_in.reshape(1, H),
                          l1['w1a'], l1['w1b'])
    ga, gb = _sc_gather(ta, tb, row3d, col3d)
    xr = jnp.take(x40, row, axis=0)
    xc = jnp.take(x40, col, axis=0)
    m, aux = _edge1(ga, gb, xr, xc, edge_attr, l1['w1e'], l1['rvec'], l1['b1'],
                    l1['w2'], l1['b2'], l1['cw1'], l1['cb1'], l1['c2w'])
    mp = _sc_scatter(m, row4d, zer)
    auxagg = jax.ops.segment_sum(aux, row, num_segments=N)
    h2, ta2, tb2, x41 = _node1(h, mp, auxagg, x40, l1['nw1a'], l1['nw1b'],
                               l1['nb1'], l1['nw2'], l1['nb2'],
                               l2['w1a'], l2['w1b'])

    ga2, gb2 = _sc_gather(ta2, tb2, row3d, col3d)
    xr2 = jnp.take(x41, row, axis=0)
    xc2 = jnp.take(x41, col, axis=0)
    m2 = _edge2(ga2, gb2, xr2, xc2, edge_attr, l2['w1e'], l2['rvec'], l2['b1'],
                l2['w2'], l2['b2'])
    mp2 = _sc_scatter(m2, row4d, zer)
    return _node_out(h2, mp2, l2['nw1a'], l2['nw1b'], l2['nb1'],
                     l2['nw2'], l2['nb2'], wo, bo.reshape(1, H))


# speedup vs baseline: 3.5470x; 2.1469x over previous
import functools

import jax
import jax.numpy as jnp
from jax import lax
from jax.experimental import pallas as pl
from jax.experimental.pallas import tpu as pltpu
from jax.experimental.pallas import tpu_sc as plsc

N = 10000
E = 320000
H = 128
ED = 16

BN = 2000
BE = 2000


def _silu(x):
    return x * jax.nn.sigmoid(x)


def _full(shape):
    return pl.BlockSpec(shape, lambda i: (0,) * len(shape))


def _rows(bsz, width):
    return pl.BlockSpec((bsz, width), lambda i: (i, 0))


def _pre_body(h0, w_in, b_in, w1a, w1b, h_out, ta_out, tb_out):
    h = jnp.dot(h0[...], w_in[...], preferred_element_type=jnp.float32) + b_in[...]
    h_out[...] = h
    ta_out[...] = jnp.dot(h, w1a[...], preferred_element_type=jnp.float32)
    tb_out[...] = jnp.dot(h, w1b[...], preferred_element_type=jnp.float32)


def _node_pre(h0, w_in, b_in, w1a, w1b):
    f32 = jnp.float32
    return pl.pallas_call(
        _pre_body,
        grid=(N // BN,),
        in_specs=[_rows(BN, H), _full((H, H)), _full((1, H)), _full((H, H)),
                  _full((H, H))],
        out_specs=[_rows(BN, H), _rows(BN, H), _rows(BN, H)],
        out_shape=[jax.ShapeDtypeStruct((N, H), f32),
                   jax.ShapeDtypeStruct((N, H), f32),
                   jax.ShapeDtypeStruct((N, H), f32)],
    )(h0, w_in, b_in, w1a, w1b)


def _edge1_body(ga, gb, drr, ea, w1e, rvec, b1r, w2, b2, cw1, cb1, c2w,
                m_out, aux_out):
    dv = drr[...]
    lane = jax.lax.broadcasted_iota(jnp.int32, dv.shape, 1)
    diff = jnp.where(lane == 3, 0.0, dv)
    radial = dv[:, 3:4]
    pre = (ga[...] + gb[...]
           + jnp.dot(ea[...], w1e[...], preferred_element_type=jnp.float32)
           + radial * rvec[...] + b1r[...])
    m1 = _silu(pre)
    m = _silu(jnp.dot(m1, w2[...], preferred_element_type=jnp.float32) + b2[...])
    cm = _silu(jnp.dot(m, cw1[...], preferred_element_type=jnp.float32) + cb1[...])
    s = jnp.sum(cm * c2w[...], axis=1, keepdims=True)
    aux = jnp.where(lane == 3, 1.0, diff * s)
    m_out[...] = m
    aux_out[...] = aux


def _edge1(ga, gb, drr, ea, w1e, rvec, b1r, w2, b2, cw1, cb1, c2w):
    f32 = jnp.float32
    return pl.pallas_call(
        _edge1_body,
        grid=(E // BE,),
        in_specs=[_rows(BE, H), _rows(BE, H), _rows(BE, 4),
                  _rows(BE, ED), _full((ED, H)), _full((1, H)), _full((1, H)),
                  _full((H, H)), _full((1, H)), _full((H, H)), _full((1, H)),
                  _full((1, H))],
        out_specs=[_rows(BE, H), _rows(BE, 4)],
        out_shape=[jax.ShapeDtypeStruct((E, H), f32),
                   jax.ShapeDtypeStruct((E, 4), f32)],
    )(ga, gb, drr, ea, w1e, rvec, b1r, w2, b2, cw1, cb1, c2w)


def _edge2_body(ga, gb, drr, ea, w1e, rvec, b1r, w2, b2, m_out):
    radial = drr[...][:, 3:4]
    pre = (ga[...] + gb[...]
           + jnp.dot(ea[...], w1e[...], preferred_element_type=jnp.float32)
           + radial * rvec[...] + b1r[...])
    m1 = _silu(pre)
    m_out[...] = _silu(jnp.dot(m1, w2[...], preferred_element_type=jnp.float32)
                       + b2[...])


def _edge2(ga, gb, drr, ea, w1e, rvec, b1r, w2, b2):
    return pl.pallas_call(
        _edge2_body,
        grid=(E // BE,),
        in_specs=[_rows(BE, H), _rows(BE, H), _rows(BE, 4),
                  _rows(BE, ED), _full((ED, H)), _full((1, H)), _full((1, H)),
                  _full((H, H)), _full((1, H))],
        out_specs=_rows(BE, H),
        out_shape=jax.ShapeDtypeStruct((E, H), jnp.float32),
    )(ga, gb, drr, ea, w1e, rvec, b1r, w2, b2)


def _node1_body(h, mp0, mp1, aux, x4, nw1a, nw1b, nb1, nw2, nb2,
                w1a, w1b, h_out, ta_out, tb_out, x_out):
    hh = h[...]
    magg = mp0[0] + mp1[0]
    av = aux[...]
    cnt = jnp.clip(av[:, 3:4], 1.0, None)
    lane = jax.lax.broadcasted_iota(jnp.int32, av.shape, 1)
    x_out[...] = x4[...] + jnp.where(lane < 3, av / cnt, 0.0)
    nh = _silu(jnp.dot(hh, nw1a[...], preferred_element_type=jnp.float32)
               + jnp.dot(magg, nw1b[...], preferred_element_type=jnp.float32)
               + nb1[...])
    h_new = hh + jnp.dot(nh, nw2[...], preferred_element_type=jnp.float32) + nb2[...]
    h_out[...] = h_new
    ta_out[...] = jnp.dot(h_new, w1a[...], preferred_element_type=jnp.float32)
    tb_out[...] = jnp.dot(h_new, w1b[...], preferred_element_type=jnp.float32)


def _part(bsz, width, p):
    return pl.BlockSpec((1, bsz, width), lambda i, _p=p: (_p, i, 0))


def _node1(h, mp, aux, x4, nw1a, nw1b, nb1, nw2, nb2, w1a, w1b):
    f32 = jnp.float32
    return pl.pallas_call(
        _node1_body,
        grid=(N // BN,),
        in_specs=[_rows(BN, H), _part(BN, H, 0), _part(BN, H, 1),
                  _rows(BN, 4), _rows(BN, 4),
                  _full((H, H)), _full((H, H)), _full((1, H)), _full((H, H)),
                  _full((1, H)), _full((H, H)), _full((H, H))],
        out_specs=[_rows(BN, H), _rows(BN, H), _rows(BN, H), _rows(BN, 4)],
        out_shape=[jax.ShapeDtypeStruct((N, H), f32),
                   jax.ShapeDtypeStruct((N, H), f32),
                   jax.ShapeDtypeStruct((N, H), f32),
                   jax.ShapeDtypeStruct((N, 4), f32)],
    )(h, mp, mp, aux, x4, nw1a, nw1b, nb1, nw2, nb2, w1a, w1b)


def _nodeout_body(h, mp0, mp1, nw1a, nw1b, nb1, nw2, nb2, wo, bo, out):
    hh = h[...]
    magg = mp0[0] + mp1[0]
    nh = _silu(jnp.dot(hh, nw1a[...], preferred_element_type=jnp.float32)
               + jnp.dot(magg, nw1b[...], preferred_element_type=jnp.float32)
               + nb1[...])
    h_new = hh + jnp.dot(nh, nw2[...], preferred_element_type=jnp.float32) + nb2[...]
    out[...] = jnp.dot(h_new, wo[...], preferred_element_type=jnp.float32) + bo[...]


def _node_out(h, mp, nw1a, nw1b, nb1, nw2, nb2, wo, bo):
    return pl.pallas_call(
        _nodeout_body,
        grid=(N // BN,),
        in_specs=[_rows(BN, H), _part(BN, H, 0), _part(BN, H, 1),
                  _full((H, H)), _full((H, H)),
                  _full((1, H)), _full((H, H)), _full((1, H)), _full((H, H)),
                  _full((1, H))],
        out_specs=_rows(BN, H),
        out_shape=jax.ShapeDtypeStruct((N, H), jnp.float32),
    )(h, mp, mp, nw1a, nw1b, nb1, nw2, nb2, wo, bo)


NS = 16
EPT = E // NS
CG = 80
NB = 5
CPT = EPT // CG
NG = CPT // NB


def _gather_body(ta_h, tb_h, r_h, c_h, ga_h, gb_h, idxv, buf, *sems):
    gsems, ssems = sems[:NB], sems[NB:]
    cid = lax.axis_index("c")
    sid = lax.axis_index("s")

    def run(tab, ih, oh):
        base = sid * EPT
        pltpu.sync_copy(ih.at[sid], idxv)

        def grp(g, _):
            handles = []
            for b in range(NB):
                j = g * NB + b

                @pl.when(g > 0)
                def _():
                    pltpu.make_async_copy(buf.at[b], oh.at[pl.ds(0, CG)],
                                          ssems[b]).wait()

                handles.append(pltpu.async_copy(tab.at[idxv.at[j]], buf.at[b],
                                                gsems[b]))
            for b in range(NB):
                off = base + (g * NB + b) * CG
                handles[b].wait()
                pltpu.async_copy(buf.at[b], oh.at[pl.ds(off, CG)], ssems[b])
            return ()

        lax.fori_loop(0, NG, grp, ())
        for b in range(NB):
            pltpu.make_async_copy(buf.at[b], oh.at[pl.ds(0, CG)],
                                  ssems[b]).wait()

    @pl.when(cid == 0)
    def _():
        run(ta_h, r_h, ga_h)

    @pl.when(cid == 1)
    def _():
        run(tb_h, c_h, gb_h)


def _sc_gather(ta, tb, row3d, col3d):
    f32 = jnp.float32
    fn = pl.kernel(
        _gather_body,
        out_type=[jax.ShapeDtypeStruct((E, H), f32),
                  jax.ShapeDtypeStruct((E, H), f32)],
        mesh=plsc.VectorSubcoreMesh(core_axis_name="c", subcore_axis_name="s"),
        scratch_types=[pltpu.VMEM((CPT, CG), jnp.int32),
                       pltpu.VMEM((NB, CG, H), f32)]
        + [pltpu.SemaphoreType.DMA] * (2 * NB),
    )
    return fn(ta, tb, row3d, col3d)


NP = 10240
RPT = NP // NS
CGS = 40
NBS = 5
CPTS = (E // 32) // CGS


NGS = CPTS // NBS


def _scatter_body(m_h, r_h, z_h, om_h, idxg, mring, accm, *sems):
    asems = sems[:NBS]
    zsem, isem = sems[NBS], sems[NBS + 1]
    cid = lax.axis_index("c")
    sid = lax.axis_index("s")

    pltpu.sync_copy(z_h, mring.at[0])
    for i in range(RPT // CGS):
        pltpu.async_copy(mring.at[0], accm.at[pl.ds(sid * RPT + i * CGS, CGS)],
                         zsem)
    for i in range(RPT // CGS):
        pltpu.make_async_copy(mring.at[0], accm.at[pl.ds(0, CGS)], zsem).wait()
    plsc.subcore_barrier()

    tid = cid * NS + sid
    base = tid * (E // 32)
    r_t = r_h.at[tid]
    pltpu.sync_copy(r_t.at[0], idxg.at[0])

    def grp(g, _):
        pb = lax.rem(g, 2)

        @pl.when(g > 0)
        def _():
            pltpu.make_async_copy(r_t.at[0], idxg.at[0], isem).wait()

        handles = []
        for b in range(NBS):
            off = base + (g * NBS + b) * CGS

            @pl.when(g > 0)
            def _():
                pltpu.make_async_copy(mring.at[b], accm.at[pl.ds(0, CGS)],
                                      asems[b]).wait()

            handles.append(pltpu.async_copy(m_h.at[pl.ds(off, CGS)],
                                            mring.at[b], asems[b]))
        for b in range(NBS):
            handles[b].wait()
            pltpu.async_copy(mring.at[b], accm.at[idxg.at[pb, b]],
                             asems[b], add=True)

        @pl.when(g + 1 < NGS)
        def _():
            pltpu.async_copy(r_t.at[g + 1], idxg.at[1 - pb], isem)

        return ()

    lax.fori_loop(0, NGS, grp, ())
    for b in range(NBS):
        pltpu.make_async_copy(mring.at[b], accm.at[pl.ds(0, CGS)],
                              asems[b]).wait()
    plsc.subcore_barrier()

    @pl.when(sid < NS - 1)
    def _():
        pltpu.sync_copy(accm.at[pl.ds(sid * RPT, RPT)],
                        om_h.at[cid].at[pl.ds(sid * RPT, RPT)])

    @pl.when(sid == NS - 1)
    def _():
        pltpu.sync_copy(accm.at[pl.ds((NS - 1) * RPT, 400)],
                        om_h.at[cid].at[pl.ds((NS - 1) * RPT, 400)])


def _sc_scatter(m, row4d, zer):
    f32 = jnp.float32
    fn = pl.kernel(
        _scatter_body,
        out_type=jax.ShapeDtypeStruct((2, N, H), f32),
        mesh=plsc.VectorSubcoreMesh(core_axis_name="c", subcore_axis_name="s"),
        scratch_types=[pltpu.VMEM((2, NBS, CGS), jnp.int32),
                       pltpu.VMEM((NBS, CGS, H), f32),
                       pltpu.VMEM_SHARED((NP, H), f32)]
        + [pltpu.SemaphoreType.DMA] * (NBS + 2),
    )
    return fn(m, row4d, zer)


NB3 = 5
CG3 = 80
CPT3 = (E // 32) // CG3


def _drr_body(x_h, r_h, c_h, o_h, xv, ridx, cidx, *rest):
    obufs, osems = rest[:NB3], rest[NB3:]
    cid = lax.axis_index("c")
    sid = lax.axis_index("s")
    tid = cid * NS + sid
    base = tid * (E // 32)
    pltpu.sync_copy(x_h, xv)
    pltpu.sync_copy(r_h.at[tid], ridx)
    pltpu.sync_copy(c_h.at[tid], cidx)
    lanes = lax.iota(jnp.int32, 16)

    def grp_fn(g, _):
        for b in range(NB3):
            j = g * NB3 + b

            @pl.when(g > 0)
            def _():
                pltpu.make_async_copy(obufs[b], o_h.at[pl.ds(0, 4 * CG3)],
                                      osems[b]).wait()

            for grp in range(CG3 // 16):
                ev = j * CG3 + grp * 16 + lanes
                rid = plsc.load_gather(ridx, [ev])
                cidv = plsc.load_gather(cidx, [ev])
                d0 = (plsc.load_gather(xv, [rid * 4])
                      - plsc.load_gather(xv, [cidv * 4]))
                d1 = (plsc.load_gather(xv, [rid * 4 + 1])
                      - plsc.load_gather(xv, [cidv * 4 + 1]))
                d2 = (plsc.load_gather(xv, [rid * 4 + 2])
                      - plsc.load_gather(xv, [cidv * 4 + 2]))
                r2 = d0 * d0 + d1 * d1 + d2 * d2
                pos = (grp * 16 + lanes) * 4
                plsc.store_scatter(obufs[b], [pos], d0)
                plsc.store_scatter(obufs[b], [pos + 1], d1)
                plsc.store_scatter(obufs[b], [pos + 2], d2)
                plsc.store_scatter(obufs[b], [pos + 3], r2)
            pltpu.async_copy(obufs[b], o_h.at[pl.ds((base + j * CG3) * 4,
                                                    4 * CG3)], osems[b])
        return ()

    lax.fori_loop(0, CPT3 // NB3, grp_fn, ())
    for b in range(NB3):
        pltpu.make_async_copy(obufs[b], o_h.at[pl.ds(0, 4 * CG3)],
                              osems[b]).wait()


def _sc_drr(x_flat, row2, col2):
    f32 = jnp.float32
    fn = pl.kernel(
        _drr_body,
        out_type=jax.ShapeDtypeStruct((E * 4,), f32),
        mesh=plsc.VectorSubcoreMesh(core_axis_name="c", subcore_axis_name="s"),
        scratch_types=[pltpu.VMEM((4 * N,), f32),
                       pltpu.VMEM((E // 32,), jnp.int32),
                       pltpu.VMEM((E // 32,), jnp.int32)]
        + [pltpu.VMEM((4 * CG3,), f32)] * NB3
        + [pltpu.SemaphoreType.DMA] * NB3,
        compiler_params=pltpu.CompilerParams(needs_layout_passes=False),
    )
    return fn(x_flat, row2, col2)


def kernel(atom_feature, coords, edge_index, edge_attr, params):
    f32 = jnp.float32
    row = edge_index[0].astype(jnp.int32)
    col = edge_index[1].astype(jnp.int32)
    x40 = jnp.pad(coords.astype(f32), ((0, 0), (0, 1)))

    w_in, b_in = params['emb_in']
    wo, bo = params['emb_out']
    lps = params['layers']

    def split_l(lp):
        w1, b1 = lp['edge1']
        return dict(
            w1a=w1[:H], w1b=w1[H:2 * H], rvec=w1[2 * H:2 * H + 1],
            w1e=w1[2 * H + 1:], b1=b1.reshape(1, H),
            w2=lp['edge2'][0], b2=lp['edge2'][1].reshape(1, H),
            cw1=lp['coord1'][0], cb1=lp['coord1'][1].reshape(1, H),
            c2w=lp['coord2_w'].reshape(1, H),
            nw1a=lp['node1'][0][:H], nw1b=lp['node1'][0][H:],
            nb1=lp['node1'][1].reshape(1, H),
            nw2=lp['node2'][0], nb2=lp['node2'][1].reshape(1, H),
        )

    l1, l2 = split_l(lps[0]), split_l(lps[1])
    row3d = row.reshape(NS, CPT, CG)
    col3d = col.reshape(NS, CPT, CG)
    row4d = row.reshape(32, NGS, NBS, CGS)
    row2 = row.reshape(32, E // 32)
    col2 = col.reshape(32, E // 32)
    zer = jnp.zeros((CGS, H), f32)

    h, ta, tb = _node_pre(atom_feature, w_in, b_in.reshape(1, H),
                          l1['w1a'], l1['w1b'])
    ga, gb = _sc_gather(ta, tb, row3d, col3d)
    drr = _sc_drr(x40.reshape(4 * N), row2, col2).reshape(E, 4)
    m, aux = _edge1(ga, gb, drr, edge_attr, l1['w1e'], l1['rvec'], l1['b1'],
                    l1['w2'], l1['b2'], l1['cw1'], l1['cb1'], l1['c2w'])
    mp = _sc_scatter(m, row4d, zer)
    auxagg = jax.ops.segment_sum(aux, row, num_segments=N)
    h2, ta2, tb2, x41 = _node1(h, mp, auxagg, x40, l1['nw1a'], l1['nw1b'],
                               l1['nb1'], l1['nw2'], l1['nb2'],
                               l2['w1a'], l2['w1b'])

    ga2, gb2 = _sc_gather(ta2, tb2, row3d, col3d)
    drr2 = _sc_drr(x41.reshape(4 * N), row2, col2).reshape(E, 4)
    m2 = _edge2(ga2, gb2, drr2, edge_attr, l2['w1e'], l2['rvec'], l2['b1'],
                l2['w2'], l2['b2'])
    mp2 = _sc_scatter(m2, row4d, zer)
    return _node_out(h2, mp2, l2['nw1a'], l2['nw1b'], l2['nb1'],
                     l2['nw2'], l2['nb2'], wo, bo.reshape(1, H))


# speedup vs baseline: 4.9262x; 1.3889x over previous
import functools

import jax
import jax.numpy as jnp
from jax import lax
from jax.experimental import pallas as pl
from jax.experimental.pallas import tpu as pltpu
from jax.experimental.pallas import tpu_sc as plsc

N = 10000
E = 320000
H = 128
ED = 16

BN = 2000
BE = 2000


def _silu(x):
    return x * jax.nn.sigmoid(x)


def _full(shape):
    return pl.BlockSpec(shape, lambda i: (0,) * len(shape))


def _rows(bsz, width):
    return pl.BlockSpec((bsz, width), lambda i: (i, 0))


def _pre_body(h0, w_in, b_in, w1a, w1b, h_out, ta_out, tb_out):
    h = jnp.dot(h0[...], w_in[...], preferred_element_type=jnp.float32) + b_in[...]
    h_out[...] = h
    ta_out[...] = jnp.dot(h, w1a[...], preferred_element_type=jnp.float32)
    tb_out[...] = jnp.dot(h, w1b[...], preferred_element_type=jnp.float32)


def _node_pre(h0, w_in, b_in, w1a, w1b):
    f32 = jnp.float32
    return pl.pallas_call(
        _pre_body,
        grid=(N // BN,),
        in_specs=[_rows(BN, H), _full((H, H)), _full((1, H)), _full((H, H)),
                  _full((H, H))],
        out_specs=[_rows(BN, H), _rows(BN, H), _rows(BN, H)],
        out_shape=[jax.ShapeDtypeStruct((N, H), f32),
                   jax.ShapeDtypeStruct((N, H), f32),
                   jax.ShapeDtypeStruct((N, H), f32)],
    )(h0, w_in, b_in, w1a, w1b)


def _edge1_body(ga, gb, drr, ea, w1e, rvec, b1r, w2, b2, cw1, cb1, c2w,
                m_out, aux_out):
    dv = drr[...]
    lane = jax.lax.broadcasted_iota(jnp.int32, dv.shape, 1)
    diff = jnp.where(lane == 3, 0.0, dv)
    radial = dv[:, 3:4]
    pre = (ga[...] + gb[...]
           + jnp.dot(ea[...], w1e[...], preferred_element_type=jnp.float32)
           + radial * rvec[...] + b1r[...])
    m1 = _silu(pre)
    m = _silu(jnp.dot(m1, w2[...], preferred_element_type=jnp.float32) + b2[...])
    cm = _silu(jnp.dot(m, cw1[...], preferred_element_type=jnp.float32) + cb1[...])
    s = jnp.sum(cm * c2w[...], axis=1, keepdims=True)
    aux = jnp.where(lane == 3, 1.0, diff * s)
    m_out[...] = m
    aux_out[...] = aux


def _edge1(ga, gb, drr, ea, w1e, rvec, b1r, w2, b2, cw1, cb1, c2w):
    f32 = jnp.float32
    return pl.pallas_call(
        _edge1_body,
        grid=(E // BE,),
        in_specs=[_rows(BE, H), _rows(BE, H), _rows(BE, 4),
                  _rows(BE, ED), _full((ED, H)), _full((1, H)), _full((1, H)),
                  _full((H, H)), _full((1, H)), _full((H, H)), _full((1, H)),
                  _full((1, H))],
        out_specs=[_rows(BE, H), _rows(BE, 4)],
        out_shape=[jax.ShapeDtypeStruct((E, H), f32),
                   jax.ShapeDtypeStruct((E, 4), f32)],
    )(ga, gb, drr, ea, w1e, rvec, b1r, w2, b2, cw1, cb1, c2w)


def _edge2_body(ga, gb, drr, ea, w1e, rvec, b1r, w2, b2, m_out):
    radial = drr[...][:, 3:4]
    pre = (ga[...] + gb[...]
           + jnp.dot(ea[...], w1e[...], preferred_element_type=jnp.float32)
           + radial * rvec[...] + b1r[...])
    m1 = _silu(pre)
    m_out[...] = _silu(jnp.dot(m1, w2[...], preferred_element_type=jnp.float32)
                       + b2[...])


def _edge2(ga, gb, drr, ea, w1e, rvec, b1r, w2, b2):
    return pl.pallas_call(
        _edge2_body,
        grid=(E // BE,),
        in_specs=[_rows(BE, H), _rows(BE, H), _rows(BE, 4),
                  _rows(BE, ED), _full((ED, H)), _full((1, H)), _full((1, H)),
                  _full((H, H)), _full((1, H))],
        out_specs=_rows(BE, H),
        out_shape=jax.ShapeDtypeStruct((E, H), jnp.float32),
    )(ga, gb, drr, ea, w1e, rvec, b1r, w2, b2)


def _node1_body(h, mp0, mp1, ap0, ap1, x4, nw1a, nw1b, nb1, nw2, nb2,
                w1a, w1b, h_out, ta_out, tb_out, x_out):
    hh = h[...]
    magg = mp0[0] + mp1[0]
    av = ap0[0] + ap1[0]
    cnt = jnp.clip(av[:, 3:4], 1.0, None)
    lane = jax.lax.broadcasted_iota(jnp.int32, av.shape, 1)
    x_out[...] = x4[...] + jnp.where(lane < 3, av / cnt, 0.0)
    nh = _silu(jnp.dot(hh, nw1a[...], preferred_element_type=jnp.float32)
               + jnp.dot(magg, nw1b[...], preferred_element_type=jnp.float32)
               + nb1[...])
    h_new = hh + jnp.dot(nh, nw2[...], preferred_element_type=jnp.float32) + nb2[...]
    h_out[...] = h_new
    ta_out[...] = jnp.dot(h_new, w1a[...], preferred_element_type=jnp.float32)
    tb_out[...] = jnp.dot(h_new, w1b[...], preferred_element_type=jnp.float32)


def _part(bsz, width, p):
    return pl.BlockSpec((1, bsz, width), lambda i, _p=p: (_p, i, 0))


def _node1(h, mp, ap, x4, nw1a, nw1b, nb1, nw2, nb2, w1a, w1b):
    f32 = jnp.float32
    return pl.pallas_call(
        _node1_body,
        grid=(N // BN,),
        in_specs=[_rows(BN, H), _part(BN, H, 0), _part(BN, H, 1),
                  _part(BN, 4, 0), _part(BN, 4, 1), _rows(BN, 4),
                  _full((H, H)), _full((H, H)), _full((1, H)), _full((H, H)),
                  _full((1, H)), _full((H, H)), _full((H, H))],
        out_specs=[_rows(BN, H), _rows(BN, H), _rows(BN, H), _rows(BN, 4)],
        out_shape=[jax.ShapeDtypeStruct((N, H), f32),
                   jax.ShapeDtypeStruct((N, H), f32),
                   jax.ShapeDtypeStruct((N, H), f32),
                   jax.ShapeDtypeStruct((N, 4), f32)],
    )(h, mp, mp, ap, ap, x4, nw1a, nw1b, nb1, nw2, nb2, w1a, w1b)


def _nodeout_body(h, mp0, mp1, nw1a, nw1b, nb1, nw2, nb2, wo, bo, out):
    hh = h[...]
    magg = mp0[0] + mp1[0]
    nh = _silu(jnp.dot(hh, nw1a[...], preferred_element_type=jnp.float32)
               + jnp.dot(magg, nw1b[...], preferred_element_type=jnp.float32)
               + nb1[...])
    h_new = hh + jnp.dot(nh, nw2[...], preferred_element_type=jnp.float32) + nb2[...]
    out[...] = jnp.dot(h_new, wo[...], preferred_element_type=jnp.float32) + bo[...]


def _node_out(h, mp, nw1a, nw1b, nb1, nw2, nb2, wo, bo):
    return pl.pallas_call(
        _nodeout_body,
        grid=(N // BN,),
        in_specs=[_rows(BN, H), _part(BN, H, 0), _part(BN, H, 1),
                  _full((H, H)), _full((H, H)),
                  _full((1, H)), _full((H, H)), _full((1, H)), _full((H, H)),
                  _full((1, H))],
        out_specs=_rows(BN, H),
        out_shape=jax.ShapeDtypeStruct((N, H), jnp.float32),
    )(h, mp, mp, nw1a, nw1b, nb1, nw2, nb2, wo, bo)


NS = 16
EPT = E // NS
CG = 80
NB = 5
CPT = EPT // CG
NG = CPT // NB


def _gather_body(ta_h, tb_h, r_h, c_h, ga_h, gb_h, idxv, buf, *sems):
    gsems, ssems = sems[:NB], sems[NB:]
    cid = lax.axis_index("c")
    sid = lax.axis_index("s")

    def run(tab, ih, oh):
        base = sid * EPT
        pltpu.sync_copy(ih.at[sid], idxv)

        def grp(g, _):
            handles = []
            for b in range(NB):
                j = g * NB + b

                @pl.when(g > 0)
                def _():
                    pltpu.make_async_copy(buf.at[b], oh.at[pl.ds(0, CG)],
                                          ssems[b]).wait()

                handles.append(pltpu.async_copy(tab.at[idxv.at[j]], buf.at[b],
                                                gsems[b]))
            for b in range(NB):
                off = base + (g * NB + b) * CG
                handles[b].wait()
                pltpu.async_copy(buf.at[b], oh.at[pl.ds(off, CG)], ssems[b])
            return ()

        lax.fori_loop(0, NG, grp, ())
        for b in range(NB):
            pltpu.make_async_copy(buf.at[b], oh.at[pl.ds(0, CG)],
                                  ssems[b]).wait()

    @pl.when(cid == 0)
    def _():
        run(ta_h, r_h, ga_h)

    @pl.when(cid == 1)
    def _():
        run(tb_h, c_h, gb_h)


def _sc_gather(ta, tb, row3d, col3d):
    f32 = jnp.float32
    fn = pl.kernel(
        _gather_body,
        out_type=[jax.ShapeDtypeStruct((E, H), f32),
                  jax.ShapeDtypeStruct((E, H), f32)],
        mesh=plsc.VectorSubcoreMesh(core_axis_name="c", subcore_axis_name="s"),
        scratch_types=[pltpu.VMEM((CPT, CG), jnp.int32),
                       pltpu.VMEM((NB, CG, H), f32)]
        + [pltpu.SemaphoreType.DMA] * (2 * NB),
    )
    return fn(ta, tb, row3d, col3d)


NP = 10240
RPT = NP // NS
CGS = 40
NBS = 5
CPTS = (E // 32) // CGS


NGS = CPTS // NBS


def _scatter_body(m_h, r_h, z_h, om_h, idxg, mring, accm, *sems):
    asems = sems[:NBS]
    zsem, isem = sems[NBS], sems[NBS + 1]
    cid = lax.axis_index("c")
    sid = lax.axis_index("s")

    pltpu.sync_copy(z_h, mring.at[0])
    for i in range(RPT // CGS):
        pltpu.async_copy(mring.at[0], accm.at[pl.ds(sid * RPT + i * CGS, CGS)],
                         zsem)
    for i in range(RPT // CGS):
        pltpu.make_async_copy(mring.at[0], accm.at[pl.ds(0, CGS)], zsem).wait()
    plsc.subcore_barrier()

    tid = cid * NS + sid
    base = tid * (E // 32)
    r_t = r_h.at[tid]
    pltpu.sync_copy(r_t.at[0], idxg.at[0])

    def grp(g, _):
        pb = lax.rem(g, 2)

        @pl.when(g > 0)
        def _():
            pltpu.make_async_copy(r_t.at[0], idxg.at[0], isem).wait()

        handles = []
        for b in range(NBS):
            off = base + (g * NBS + b) * CGS

            @pl.when(g > 0)
            def _():
                pltpu.make_async_copy(mring.at[b], accm.at[pl.ds(0, CGS)],
                                      asems[b]).wait()

            handles.append(pltpu.async_copy(m_h.at[pl.ds(off, CGS)],
                                            mring.at[b], asems[b]))
        for b in range(NBS):
            handles[b].wait()
            pltpu.async_copy(mring.at[b], accm.at[idxg.at[pb, b]],
                             asems[b], add=True)

        @pl.when(g + 1 < NGS)
        def _():
            pltpu.async_copy(r_t.at[g + 1], idxg.at[1 - pb], isem)

        return ()

    lax.fori_loop(0, NGS, grp, ())
    for b in range(NBS):
        pltpu.make_async_copy(mring.at[b], accm.at[pl.ds(0, CGS)],
                              asems[b]).wait()
    plsc.subcore_barrier()

    @pl.when(sid < NS - 1)
    def _():
        pltpu.sync_copy(accm.at[pl.ds(sid * RPT, RPT)],
                        om_h.at[cid].at[pl.ds(sid * RPT, RPT)])

    @pl.when(sid == NS - 1)
    def _():
        pltpu.sync_copy(accm.at[pl.ds((NS - 1) * RPT, 400)],
                        om_h.at[cid].at[pl.ds((NS - 1) * RPT, 400)])


def _sc_scatter(m, row4d, zer):
    f32 = jnp.float32
    fn = pl.kernel(
        _scatter_body,
        out_type=jax.ShapeDtypeStruct((2, N, H), f32),
        mesh=plsc.VectorSubcoreMesh(core_axis_name="c", subcore_axis_name="s"),
        scratch_types=[pltpu.VMEM((2, NBS, CGS), jnp.int32),
                       pltpu.VMEM((NBS, CGS, H), f32),
                       pltpu.VMEM_SHARED((NP, H), f32)]
        + [pltpu.SemaphoreType.DMA] * (NBS + 2),
    )
    return fn(m, row4d, zer)


def _scatter_aux_body(m_h, a_h, r_h, r2_h, z_h, z1_h, om_h, oa_h,
                      idxg, idxf, mring, accm, acc4, zb4, *rest):
    vbufs = rest[:NBS]
    eb0 = rest[NBS:2 * NBS]
    eb1 = rest[2 * NBS:3 * NBS]
    sems = rest[3 * NBS:]
    asems = sems[:NBS]
    zsem, isem = sems[NBS], sems[NBS + 1]
    cid = lax.axis_index("c")
    sid = lax.axis_index("s")

    pltpu.sync_copy(z_h, mring.at[0])
    pltpu.sync_copy(z1_h, zb4)
    for i in range(RPT // CGS):
        pltpu.async_copy(mring.at[0], accm.at[pl.ds(sid * RPT + i * CGS, CGS)],
                         zsem)
    pltpu.async_copy(zb4, acc4.at[pl.ds(sid * 4 * RPT, 4 * RPT)], zsem)
    for i in range(RPT // CGS):
        pltpu.make_async_copy(mring.at[0], accm.at[pl.ds(0, CGS)], zsem).wait()
    pltpu.make_async_copy(zb4, acc4.at[pl.ds(0, 4 * RPT)], zsem).wait()
    plsc.subcore_barrier()

    tid = cid * NS + sid
    base = tid * (E // 32)
    r_t = r_h.at[tid]
    pltpu.sync_copy(r_t.at[0], idxg.at[0])
    pltpu.sync_copy(r2_h.at[tid], idxf)

    def grp(g, _):
        pb = lax.rem(g, 2)

        @pl.when(g > 0)
        def _():
            pltpu.make_async_copy(r_t.at[0], idxg.at[0], isem).wait()

        handles = []
        for b in range(NBS):
            off = base + (g * NBS + b) * CGS

            @pl.when(g > 0)
            def _():
                pltpu.make_async_copy(mring.at[b], accm.at[pl.ds(0, CGS)],
                                      asems[b]).wait()
                pltpu.make_async_copy(vbufs[b].at[pl.ds(0, 2 * CGS)],
                                      acc4.at[pl.ds(0, 2 * CGS)],
                                      asems[b]).wait()
                pltpu.make_async_copy(vbufs[b].at[pl.ds(0, 2 * CGS)],
                                      acc4.at[pl.ds(0, 2 * CGS)],
                                      asems[b]).wait()

            handles.append(pltpu.async_copy(m_h.at[pl.ds(off, CGS)],
                                            mring.at[b], asems[b]))
            handles.append(pltpu.async_copy(a_h.at[pl.ds(off * 4, 4 * CGS)],
                                            vbufs[b], asems[b]))
        for b in range(NBS):
            j = g * NBS + b
            handles[2 * b].wait()
            handles[2 * b + 1].wait()
            for g8 in range(4 * CGS // 16):
                lanes = lax.iota(jnp.int32, 16)
                e = j * CGS + g8 * 4 + lanes // 4
                rv = plsc.load_gather(idxf, [e])
                el = rv * 4 + lax.rem(lanes, 4)
                if g8 < 2 * CGS // 16:
                    plsc.store_scatter(eb0[b], [g8 * 16 + lanes], el)
                else:
                    plsc.store_scatter(eb1[b],
                                       [(g8 - 2 * CGS // 16) * 16 + lanes], el)
            pltpu.async_copy(mring.at[b], accm.at[idxg.at[pb, b]],
                             asems[b], add=True)
            pltpu.async_copy(vbufs[b].at[pl.ds(0, 2 * CGS)], acc4.at[eb0[b]],
                             asems[b], add=True)
            pltpu.async_copy(vbufs[b].at[pl.ds(2 * CGS, 2 * CGS)],
                             acc4.at[eb1[b]], asems[b], add=True)

        @pl.when(g + 1 < NGS)
        def _():
            pltpu.async_copy(r_t.at[g + 1], idxg.at[1 - pb], isem)

        return ()

    lax.fori_loop(0, NGS, grp, ())
    for b in range(NBS):
        pltpu.make_async_copy(mring.at[b], accm.at[pl.ds(0, CGS)],
                              asems[b]).wait()
        pltpu.make_async_copy(vbufs[b].at[pl.ds(0, 2 * CGS)],
                              acc4.at[pl.ds(0, 2 * CGS)], asems[b]).wait()
        pltpu.make_async_copy(vbufs[b].at[pl.ds(0, 2 * CGS)],
                              acc4.at[pl.ds(0, 2 * CGS)], asems[b]).wait()
    plsc.subcore_barrier()

    pltpu.sync_copy(acc4.at[pl.ds(sid * 4 * RPT, 4 * RPT)],
                    oa_h.at[cid].at[pl.ds(sid * 4 * RPT, 4 * RPT)])

    @pl.when(sid < NS - 1)
    def _():
        pltpu.sync_copy(accm.at[pl.ds(sid * RPT, RPT)],
                        om_h.at[cid].at[pl.ds(sid * RPT, RPT)])

    @pl.when(sid == NS - 1)
    def _():
        pltpu.sync_copy(accm.at[pl.ds((NS - 1) * RPT, 400)],
                        om_h.at[cid].at[pl.ds((NS - 1) * RPT, 400)])


def _sc_scatter_aux(m, aux_flat, row4d, row2, zer, zer1):
    f32 = jnp.float32
    fn = pl.kernel(
        _scatter_aux_body,
        out_type=[jax.ShapeDtypeStruct((2, N, H), f32),
                  jax.ShapeDtypeStruct((2, 4 * NP), f32)],
        mesh=plsc.VectorSubcoreMesh(core_axis_name="c", subcore_axis_name="s"),
        scratch_types=[pltpu.VMEM((2, NBS, CGS), jnp.int32),
                       pltpu.VMEM((E // 32,), jnp.int32),
                       pltpu.VMEM((NBS, CGS, H), f32),
                       pltpu.VMEM_SHARED((NP, H), f32),
                       pltpu.VMEM_SHARED((4 * NP,), f32),
                       pltpu.VMEM((4 * RPT,), f32)]
        + [pltpu.VMEM((4 * CGS,), f32)] * NBS
        + [pltpu.VMEM((2 * CGS,), jnp.int32)] * (2 * NBS)
        + [pltpu.SemaphoreType.DMA] * (NBS + 2),
        compiler_params=pltpu.CompilerParams(needs_layout_passes=False),
    )
    return fn(m, aux_flat, row4d, row2, zer, zer1)


NB3 = 5
CG3 = 80
CPT3 = (E // 32) // CG3


def _drr_body(x_h, r_h, c_h, o_h, xv, ridx, cidx, *rest):
    obufs, osems = rest[:NB3], rest[NB3:]
    cid = lax.axis_index("c")
    sid = lax.axis_index("s")
    tid = cid * NS + sid
    base = tid * (E // 32)
    pltpu.sync_copy(x_h, xv)
    pltpu.sync_copy(r_h.at[tid], ridx)
    pltpu.sync_copy(c_h.at[tid], cidx)
    lanes = lax.iota(jnp.int32, 16)

    def grp_fn(g, _):
        for b in range(NB3):
            j = g * NB3 + b

            @pl.when(g > 0)
            def _():
                pltpu.make_async_copy(obufs[b], o_h.at[pl.ds(0, 4 * CG3)],
                                      osems[b]).wait()

            for grp in range(CG3 // 16):
                ev = j * CG3 + grp * 16 + lanes
                rid = plsc.load_gather(ridx, [ev])
                cidv = plsc.load_gather(cidx, [ev])
                d0 = (plsc.load_gather(xv, [rid * 4])
                      - plsc.load_gather(xv, [cidv * 4]))
                d1 = (plsc.load_gather(xv, [rid * 4 + 1])
                      - plsc.load_gather(xv, [cidv * 4 + 1]))
                d2 = (plsc.load_gather(xv, [rid * 4 + 2])
                      - plsc.load_gather(xv, [cidv * 4 + 2]))
                r2 = d0 * d0 + d1 * d1 + d2 * d2
                pos = (grp * 16 + lanes) * 4
                plsc.store_scatter(obufs[b], [pos], d0)
                plsc.store_scatter(obufs[b], [pos + 1], d1)
                plsc.store_scatter(obufs[b], [pos + 2], d2)
                plsc.store_scatter(obufs[b], [pos + 3], r2)
            pltpu.async_copy(obufs[b], o_h.at[pl.ds((base + j * CG3) * 4,
                                                    4 * CG3)], osems[b])
        return ()

    lax.fori_loop(0, CPT3 // NB3, grp_fn, ())
    for b in range(NB3):
        pltpu.make_async_copy(obufs[b], o_h.at[pl.ds(0, 4 * CG3)],
                              osems[b]).wait()


def _sc_drr(x_flat, row2, col2):
    f32 = jnp.float32
    fn = pl.kernel(
        _drr_body,
        out_type=jax.ShapeDtypeStruct((E * 4,), f32),
        mesh=plsc.VectorSubcoreMesh(core_axis_name="c", subcore_axis_name="s"),
        scratch_types=[pltpu.VMEM((4 * N,), f32),
                       pltpu.VMEM((E // 32,), jnp.int32),
                       pltpu.VMEM((E // 32,), jnp.int32)]
        + [pltpu.VMEM((4 * CG3,), f32)] * NB3
        + [pltpu.SemaphoreType.DMA] * NB3,
        compiler_params=pltpu.CompilerParams(needs_layout_passes=False),
    )
    return fn(x_flat, row2, col2)


def kernel(atom_feature, coords, edge_index, edge_attr, params):
    f32 = jnp.float32
    row = edge_index[0].astype(jnp.int32)
    col = edge_index[1].astype(jnp.int32)
    x40 = jnp.pad(coords.astype(f32), ((0, 0), (0, 1)))

    w_in, b_in = params['emb_in']
    wo, bo = params['emb_out']
    lps = params['layers']

    def split_l(lp):
        w1, b1 = lp['edge1']
        return dict(
            w1a=w1[:H], w1b=w1[H:2 * H], rvec=w1[2 * H:2 * H + 1],
            w1e=w1[2 * H + 1:], b1=b1.reshape(1, H),
            w2=lp['edge2'][0], b2=lp['edge2'][1].reshape(1, H),
            cw1=lp['coord1'][0], cb1=lp['coord1'][1].reshape(1, H),
            c2w=lp['coord2_w'].reshape(1, H),
            nw1a=lp['node1'][0][:H], nw1b=lp['node1'][0][H:],
            nb1=lp['node1'][1].reshape(1, H),
            nw2=lp['node2'][0], nb2=lp['node2'][1].reshape(1, H),
        )

    l1, l2 = split_l(lps[0]), split_l(lps[1])
    row3d = row.reshape(NS, CPT, CG)
    col3d = col.reshape(NS, CPT, CG)
    row4d = row.reshape(32, NGS, NBS, CGS)
    row2 = row.reshape(32, E // 32)
    col2 = col.reshape(32, E // 32)
    zer = jnp.zeros((CGS, H), f32)
    zer1 = jnp.zeros((4 * RPT,), f32)

    h, ta, tb = _node_pre(atom_feature, w_in, b_in.reshape(1, H),
                          l1['w1a'], l1['w1b'])
    ga, gb = _sc_gather(ta, tb, row3d, col3d)
    drr = _sc_drr(x40.reshape(4 * N), row2, col2).reshape(E, 4)
    m, aux = _edge1(ga, gb, drr, edge_attr, l1['w1e'], l1['rvec'], l1['b1'],
                    l1['w2'], l1['b2'], l1['cw1'], l1['cb1'], l1['c2w'])
    mp, oa = _sc_scatter_aux(m, aux.reshape(4 * E), row4d, row2, zer, zer1)
    ap = oa.reshape(2, NP, 4)
    h2, ta2, tb2, x41 = _node1(h, mp, ap, x40, l1['nw1a'], l1['nw1b'],
                               l1['nb1'], l1['nw2'], l1['nb2'],
                               l2['w1a'], l2['w1b'])

    ga2, gb2 = _sc_gather(ta2, tb2, row3d, col3d)
    drr2 = _sc_drr(x41.reshape(4 * N), row2, col2).reshape(E, 4)
    m2 = _edge2(ga2, gb2, drr2, edge_attr, l2['w1e'], l2['rvec'], l2['b1'],
                l2['w2'], l2['b2'])
    mp2 = _sc_scatter(m2, row4d, zer)
    return _node_out(h2, mp2, l2['nw1a'], l2['nw1b'], l2['nb1'],
                     l2['nw2'], l2['nb2'], wo, bo.reshape(1, H))


# speedup vs baseline: 5.0952x; 1.0343x over previous
import functools

import jax
import jax.numpy as jnp
from jax import lax
from jax.experimental import pallas as pl
from jax.experimental.pallas import tpu as pltpu
from jax.experimental.pallas import tpu_sc as plsc

N = 10000
E = 320000
H = 128
ED = 16

BN = 2000
BE = 4000


def _silu(x):
    return x * jax.nn.sigmoid(x)


def _full(shape):
    return pl.BlockSpec(shape, lambda i: (0,) * len(shape))


def _rows(bsz, width):
    return pl.BlockSpec((bsz, width), lambda i: (i, 0))


def _pre_body(h0, w_in, b_in, w1a, w1b, h_out, ta_out, tb_out):
    h = jnp.dot(h0[...], w_in[...], preferred_element_type=jnp.float32) + b_in[...]
    h_out[...] = h
    ta_out[...] = jnp.dot(h, w1a[...], preferred_element_type=jnp.float32)
    tb_out[...] = jnp.dot(h, w1b[...], preferred_element_type=jnp.float32)


def _node_pre(h0, w_in, b_in, w1a, w1b):
    f32 = jnp.float32
    return pl.pallas_call(
        _pre_body,
        grid=(N // BN,),
        in_specs=[_rows(BN, H), _full((H, H)), _full((1, H)), _full((H, H)),
                  _full((H, H))],
        out_specs=[_rows(BN, H), _rows(BN, H), _rows(BN, H)],
        out_shape=[jax.ShapeDtypeStruct((N, H), f32),
                   jax.ShapeDtypeStruct((N, H), f32),
                   jax.ShapeDtypeStruct((N, H), f32)],
    )(h0, w_in, b_in, w1a, w1b)


def _edge1_body(ga, gb, drr, ea, w1e, rvec, b1r, w2, b2, cw1, cb1, c2w,
                m_out, aux_out):
    dv = drr[...]
    lane = jax.lax.broadcasted_iota(jnp.int32, dv.shape, 1)
    diff = jnp.where(lane == 3, 0.0, dv)
    radial = dv[:, 3:4]
    pre = (ga[...] + gb[...]
           + jnp.dot(ea[...], w1e[...], preferred_element_type=jnp.float32)
           + radial * rvec[...] + b1r[...])
    m1 = _silu(pre)
    m = _silu(jnp.dot(m1, w2[...], preferred_element_type=jnp.float32) + b2[...])
    cm = _silu(jnp.dot(m, cw1[...], preferred_element_type=jnp.float32) + cb1[...])
    s = jnp.sum(cm * c2w[...], axis=1, keepdims=True)
    aux = jnp.where(lane == 3, 1.0, diff * s)
    m_out[...] = m
    aux_out[...] = aux


def _edge1(ga, gb, drr, ea, w1e, rvec, b1r, w2, b2, cw1, cb1, c2w):
    f32 = jnp.float32
    return pl.pallas_call(
        _edge1_body,
        grid=(E // BE,),
        in_specs=[_rows(BE, H), _rows(BE, H), _rows(BE, 4),
                  _rows(BE, ED), _full((ED, H)), _full((1, H)), _full((1, H)),
                  _full((H, H)), _full((1, H)), _full((H, H)), _full((1, H)),
                  _full((1, H))],
        out_specs=[_rows(BE, H), _rows(BE, 4)],
        out_shape=[jax.ShapeDtypeStruct((E, H), f32),
                   jax.ShapeDtypeStruct((E, 4), f32)],
    )(ga, gb, drr, ea, w1e, rvec, b1r, w2, b2, cw1, cb1, c2w)


def _edge2_body(ga, gb, drr, ea, w1e, rvec, b1r, w2, b2, m_out):
    radial = drr[...][:, 3:4]
    pre = (ga[...] + gb[...]
           + jnp.dot(ea[...], w1e[...], preferred_element_type=jnp.float32)
           + radial * rvec[...] + b1r[...])
    m1 = _silu(pre)
    m_out[...] = _silu(jnp.dot(m1, w2[...], preferred_element_type=jnp.float32)
                       + b2[...])


def _edge2(ga, gb, drr, ea, w1e, rvec, b1r, w2, b2):
    return pl.pallas_call(
        _edge2_body,
        grid=(E // BE,),
        in_specs=[_rows(BE, H), _rows(BE, H), _rows(BE, 4),
                  _rows(BE, ED), _full((ED, H)), _full((1, H)), _full((1, H)),
                  _full((H, H)), _full((1, H))],
        out_specs=_rows(BE, H),
        out_shape=jax.ShapeDtypeStruct((E, H), jnp.float32),
    )(ga, gb, drr, ea, w1e, rvec, b1r, w2, b2)


def _node1_body(h, mp0, mp1, ap0, ap1, x4, nw1a, nw1b, nb1, nw2, nb2,
                w1a, w1b, h_out, ta_out, tb_out, x_out):
    hh = h[...]
    magg = mp0[0] + mp1[0]
    av = ap0[0] + ap1[0]
    cnt = jnp.clip(av[:, 3:4], 1.0, None)
    lane = jax.lax.broadcasted_iota(jnp.int32, av.shape, 1)
    x_out[...] = x4[...] + jnp.where(lane < 3, av / cnt, 0.0)
    nh = _silu(jnp.dot(hh, nw1a[...], preferred_element_type=jnp.float32)
               + jnp.dot(magg, nw1b[...], preferred_element_type=jnp.float32)
               + nb1[...])
    h_new = hh + jnp.dot(nh, nw2[...], preferred_element_type=jnp.float32) + nb2[...]
    h_out[...] = h_new
    ta_out[...] = jnp.dot(h_new, w1a[...], preferred_element_type=jnp.float32)
    tb_out[...] = jnp.dot(h_new, w1b[...], preferred_element_type=jnp.float32)


def _part(bsz, width, p):
    return pl.BlockSpec((1, bsz, width), lambda i, _p=p: (_p, i, 0))


def _node1(h, mp, ap, x4, nw1a, nw1b, nb1, nw2, nb2, w1a, w1b):
    f32 = jnp.float32
    return pl.pallas_call(
        _node1_body,
        grid=(N // BN,),
        in_specs=[_rows(BN, H), _part(BN, H, 0), _part(BN, H, 1),
                  _part(BN, 4, 0), _part(BN, 4, 1), _rows(BN, 4),
                  _full((H, H)), _full((H, H)), _full((1, H)), _full((H, H)),
                  _full((1, H)), _full((H, H)), _full((H, H))],
        out_specs=[_rows(BN, H), _rows(BN, H), _rows(BN, H), _rows(BN, 4)],
        out_shape=[jax.ShapeDtypeStruct((N, H), f32),
                   jax.ShapeDtypeStruct((N, H), f32),
                   jax.ShapeDtypeStruct((N, H), f32),
                   jax.ShapeDtypeStruct((N, 4), f32)],
    )(h, mp, mp, ap, ap, x4, nw1a, nw1b, nb1, nw2, nb2, w1a, w1b)


def _nodeout_body(h, mp0, mp1, nw1a, nw1b, nb1, nw2, nb2, wo, bo, out):
    hh = h[...]
    magg = mp0[0] + mp1[0]
    nh = _silu(jnp.dot(hh, nw1a[...], preferred_element_type=jnp.float32)
               + jnp.dot(magg, nw1b[...], preferred_element_type=jnp.float32)
               + nb1[...])
    h_new = hh + jnp.dot(nh, nw2[...], preferred_element_type=jnp.float32) + nb2[...]
    out[...] = jnp.dot(h_new, wo[...], preferred_element_type=jnp.float32) + bo[...]


def _node_out(h, mp, nw1a, nw1b, nb1, nw2, nb2, wo, bo):
    return pl.pallas_call(
        _nodeout_body,
        grid=(N // BN,),
        in_specs=[_rows(BN, H), _part(BN, H, 0), _part(BN, H, 1),
                  _full((H, H)), _full((H, H)),
                  _full((1, H)), _full((H, H)), _full((1, H)), _full((H, H)),
                  _full((1, H))],
        out_specs=_rows(BN, H),
        out_shape=jax.ShapeDtypeStruct((N, H), jnp.float32),
    )(h, mp, mp, nw1a, nw1b, nb1, nw2, nb2, wo, bo)


NS = 16
EPT = E // NS
CG = 80
NB = 5
CPT = EPT // CG
NG = CPT // NB


def _gather_body(ta_h, tb_h, r_h, c_h, ga_h, gb_h, idxv, buf, *sems):
    gsems, ssems = sems[:NB], sems[NB:]
    cid = lax.axis_index("c")
    sid = lax.axis_index("s")

    def run(tab, ih, oh):
        base = sid * EPT
        pltpu.sync_copy(ih.at[sid], idxv)

        def grp(g, _):
            handles = []
            for b in range(NB):
                j = g * NB + b

                @pl.when(g > 0)
                def _():
                    pltpu.make_async_copy(buf.at[b], oh.at[pl.ds(0, CG)],
                                          ssems[b]).wait()

                handles.append(pltpu.async_copy(tab.at[idxv.at[j]], buf.at[b],
                                                gsems[b]))
            for b in range(NB):
                off = base + (g * NB + b) * CG
                handles[b].wait()
                pltpu.async_copy(buf.at[b], oh.at[pl.ds(off, CG)], ssems[b])
            return ()

        lax.fori_loop(0, NG, grp, ())
        for b in range(NB):
            pltpu.make_async_copy(buf.at[b], oh.at[pl.ds(0, CG)],
                                  ssems[b]).wait()

    @pl.when(cid == 0)
    def _():
        run(ta_h, r_h, ga_h)

    @pl.when(cid == 1)
    def _():
        run(tb_h, c_h, gb_h)


def _sc_gather(ta, tb, row3d, col3d):
    f32 = jnp.float32
    fn = pl.kernel(
        _gather_body,
        out_type=[jax.ShapeDtypeStruct((E, H), f32),
                  jax.ShapeDtypeStruct((E, H), f32)],
        mesh=plsc.VectorSubcoreMesh(core_axis_name="c", subcore_axis_name="s"),
        scratch_types=[pltpu.VMEM((CPT, CG), jnp.int32),
                       pltpu.VMEM((NB, CG, H), f32)]
        + [pltpu.SemaphoreType.DMA] * (2 * NB),
    )
    return fn(ta, tb, row3d, col3d)


NP = 10240
RPT = NP // NS
CGS = 40
NBS = 5
CPTS = (E // 32) // CGS


NGS = CPTS // NBS


def _scatter_body(m_h, r_h, z_h, om_h, idxg, mring, accm, *sems):
    asems = sems[:NBS]
    zsem, isem = sems[NBS], sems[NBS + 1]
    cid = lax.axis_index("c")
    sid = lax.axis_index("s")

    pltpu.sync_copy(z_h, mring.at[0])
    for i in range(RPT // CGS):
        pltpu.async_copy(mring.at[0], accm.at[pl.ds(sid * RPT + i * CGS, CGS)],
                         zsem)
    for i in range(RPT // CGS):
        pltpu.make_async_copy(mring.at[0], accm.at[pl.ds(0, CGS)], zsem).wait()
    plsc.subcore_barrier()

    tid = cid * NS + sid
    base = tid * (E // 32)
    r_t = r_h.at[tid]
    pltpu.sync_copy(r_t.at[0], idxg.at[0])

    def grp(g, _):
        pb = lax.rem(g, 2)

        @pl.when(g > 0)
        def _():
            pltpu.make_async_copy(r_t.at[0], idxg.at[0], isem).wait()

        handles = []
        for b in range(NBS):
            off = base + (g * NBS + b) * CGS

            @pl.when(g > 0)
            def _():
                pltpu.make_async_copy(mring.at[b], accm.at[pl.ds(0, CGS)],
                                      asems[b]).wait()

            handles.append(pltpu.async_copy(m_h.at[pl.ds(off, CGS)],
                                            mring.at[b], asems[b]))
        for b in range(NBS):
            handles[b].wait()
            pltpu.async_copy(mring.at[b], accm.at[idxg.at[pb, b]],
                             asems[b], add=True)

        @pl.when(g + 1 < NGS)
        def _():
            pltpu.async_copy(r_t.at[g + 1], idxg.at[1 - pb], isem)

        return ()

    lax.fori_loop(0, NGS, grp, ())
    for b in range(NBS):
        pltpu.make_async_copy(mring.at[b], accm.at[pl.ds(0, CGS)],
                              asems[b]).wait()
    plsc.subcore_barrier()

    @pl.when(sid < NS - 1)
    def _():
        pltpu.sync_copy(accm.at[pl.ds(sid * RPT, RPT)],
                        om_h.at[cid].at[pl.ds(sid * RPT, RPT)])

    @pl.when(sid == NS - 1)
    def _():
        pltpu.sync_copy(accm.at[pl.ds((NS - 1) * RPT, 400)],
                        om_h.at[cid].at[pl.ds((NS - 1) * RPT, 400)])


def _sc_scatter(m, row4d, zer):
    f32 = jnp.float32
    fn = pl.kernel(
        _scatter_body,
        out_type=jax.ShapeDtypeStruct((2, N, H), f32),
        mesh=plsc.VectorSubcoreMesh(core_axis_name="c", subcore_axis_name="s"),
        scratch_types=[pltpu.VMEM((2, NBS, CGS), jnp.int32),
                       pltpu.VMEM((NBS, CGS, H), f32),
                       pltpu.VMEM_SHARED((NP, H), f32)]
        + [pltpu.SemaphoreType.DMA] * (NBS + 2),
    )
    return fn(m, row4d, zer)


def _scatter_aux_body(m_h, a_h, r_h, r2_h, z_h, z1_h, om_h, oa_h,
                      idxg, idxf, mring, accm, acc4, zb4, *rest):
    vbufs = rest[:NBS]
    eb0 = rest[NBS:2 * NBS]
    eb1 = rest[2 * NBS:3 * NBS]
    sems = rest[3 * NBS:]
    asems = sems[:NBS]
    zsem, isem = sems[NBS], sems[NBS + 1]
    cid = lax.axis_index("c")
    sid = lax.axis_index("s")

    pltpu.sync_copy(z_h, mring.at[0])
    pltpu.sync_copy(z1_h, zb4)
    for i in range(RPT // CGS):
        pltpu.async_copy(mring.at[0], accm.at[pl.ds(sid * RPT + i * CGS, CGS)],
                         zsem)
    pltpu.async_copy(zb4, acc4.at[pl.ds(sid * 4 * RPT, 4 * RPT)], zsem)
    for i in range(RPT // CGS):
        pltpu.make_async_copy(mring.at[0], accm.at[pl.ds(0, CGS)], zsem).wait()
    pltpu.make_async_copy(zb4, acc4.at[pl.ds(0, 4 * RPT)], zsem).wait()
    plsc.subcore_barrier()

    tid = cid * NS + sid
    base = tid * (E // 32)
    r_t = r_h.at[tid]
    pltpu.sync_copy(r_t.at[0], idxg.at[0])
    pltpu.sync_copy(r2_h.at[tid], idxf)

    def grp(g, _):
        pb = lax.rem(g, 2)

        @pl.when(g > 0)
        def _():
            pltpu.make_async_copy(r_t.at[0], idxg.at[0], isem).wait()

        handles = []
        for b in range(NBS):
            off = base + (g * NBS + b) * CGS

            @pl.when(g > 0)
            def _():
                pltpu.make_async_copy(mring.at[b], accm.at[pl.ds(0, CGS)],
                                      asems[b]).wait()
                pltpu.make_async_copy(vbufs[b].at[pl.ds(0, 2 * CGS)],
                                      acc4.at[pl.ds(0, 2 * CGS)],
                                      asems[b]).wait()
                pltpu.make_async_copy(vbufs[b].at[pl.ds(0, 2 * CGS)],
                                      acc4.at[pl.ds(0, 2 * CGS)],
                                      asems[b]).wait()

            handles.append(pltpu.async_copy(m_h.at[pl.ds(off, CGS)],
                                            mring.at[b], asems[b]))
            handles.append(pltpu.async_copy(a_h.at[pl.ds(off * 4, 4 * CGS)],
                                            vbufs[b], asems[b]))
        for b in range(NBS):
            j = g * NBS + b
            handles[2 * b].wait()
            handles[2 * b + 1].wait()
            for g8 in range(4 * CGS // 16):
                lanes = lax.iota(jnp.int32, 16)
                e = j * CGS + g8 * 4 + lanes // 4
                rv = plsc.load_gather(idxf, [e])
                el = rv * 4 + lax.rem(lanes, 4)
                if g8 < 2 * CGS // 16:
                    plsc.store_scatter(eb0[b], [g8 * 16 + lanes], el)
                else:
                    plsc.store_scatter(eb1[b],
                                       [(g8 - 2 * CGS // 16) * 16 + lanes], el)
            pltpu.async_copy(mring.at[b], accm.at[idxg.at[pb, b]],
                             asems[b], add=True)
            pltpu.async_copy(vbufs[b].at[pl.ds(0, 2 * CGS)], acc4.at[eb0[b]],
                             asems[b], add=True)
            pltpu.async_copy(vbufs[b].at[pl.ds(2 * CGS, 2 * CGS)],
                             acc4.at[eb1[b]], asems[b], add=True)

        @pl.when(g + 1 < NGS)
        def _():
            pltpu.async_copy(r_t.at[g + 1], idxg.at[1 - pb], isem)

        return ()

    lax.fori_loop(0, NGS, grp, ())
    for b in range(NBS):
        pltpu.make_async_copy(mring.at[b], accm.at[pl.ds(0, CGS)],
                              asems[b]).wait()
        pltpu.make_async_copy(vbufs[b].at[pl.ds(0, 2 * CGS)],
                              acc4.at[pl.ds(0, 2 * CGS)], asems[b]).wait()
        pltpu.make_async_copy(vbufs[b].at[pl.ds(0, 2 * CGS)],
                              acc4.at[pl.ds(0, 2 * CGS)], asems[b]).wait()
    plsc.subcore_barrier()

    pltpu.sync_copy(acc4.at[pl.ds(sid * 4 * RPT, 4 * RPT)],
                    oa_h.at[cid].at[pl.ds(sid * 4 * RPT, 4 * RPT)])

    @pl.when(sid < NS - 1)
    def _():
        pltpu.sync_copy(accm.at[pl.ds(sid * RPT, RPT)],
                        om_h.at[cid].at[pl.ds(sid * RPT, RPT)])

    @pl.when(sid == NS - 1)
    def _():
        pltpu.sync_copy(accm.at[pl.ds((NS - 1) * RPT, 400)],
                        om_h.at[cid].at[pl.ds((NS - 1) * RPT, 400)])


def _sc_scatter_aux(m, aux_flat, row4d, row2, zer, zer1):
    f32 = jnp.float32
    fn = pl.kernel(
        _scatter_aux_body,
        out_type=[jax.ShapeDtypeStruct((2, N, H), f32),
                  jax.ShapeDtypeStruct((2, 4 * NP), f32)],
        mesh=plsc.VectorSubcoreMesh(core_axis_name="c", subcore_axis_name="s"),
        scratch_types=[pltpu.VMEM((2, NBS, CGS), jnp.int32),
                       pltpu.VMEM((E // 32,), jnp.int32),
                       pltpu.VMEM((NBS, CGS, H), f32),
                       pltpu.VMEM_SHARED((NP, H), f32),
                       pltpu.VMEM_SHARED((4 * NP,), f32),
                       pltpu.VMEM((4 * RPT,), f32)]
        + [pltpu.VMEM((4 * CGS,), f32)] * NBS
        + [pltpu.VMEM((2 * CGS,), jnp.int32)] * (2 * NBS)
        + [pltpu.SemaphoreType.DMA] * (NBS + 2),
        compiler_params=pltpu.CompilerParams(needs_layout_passes=False),
    )
    return fn(m, aux_flat, row4d, row2, zer, zer1)


NB3 = 5
CG3 = 80
CPT3 = (E // 32) // CG3


def _drr_body(x_h, r_h, c_h, o_h, xv, ridx, cidx, *rest):
    obufs, osems = rest[:NB3], rest[NB3:]
    cid = lax.axis_index("c")
    sid = lax.axis_index("s")
    tid = cid * NS + sid
    base = tid * (E // 32)
    pltpu.sync_copy(x_h, xv)
    pltpu.sync_copy(r_h.at[tid], ridx)
    pltpu.sync_copy(c_h.at[tid], cidx)
    lanes = lax.iota(jnp.int32, 16)

    def grp_fn(g, _):
        for b in range(NB3):
            j = g * NB3 + b

            @pl.when(g > 0)
            def _():
                pltpu.make_async_copy(obufs[b], o_h.at[pl.ds(0, 4 * CG3)],
                                      osems[b]).wait()

            for grp in range(CG3 // 16):
                ev = j * CG3 + grp * 16 + lanes
                rid = plsc.load_gather(ridx, [ev])
                cidv = plsc.load_gather(cidx, [ev])
                d0 = (plsc.load_gather(xv, [rid * 4])
                      - plsc.load_gather(xv, [cidv * 4]))
                d1 = (plsc.load_gather(xv, [rid * 4 + 1])
                      - plsc.load_gather(xv, [cidv * 4 + 1]))
                d2 = (plsc.load_gather(xv, [rid * 4 + 2])
                      - plsc.load_gather(xv, [cidv * 4 + 2]))
                r2 = d0 * d0 + d1 * d1 + d2 * d2
                pos = (grp * 16 + lanes) * 4
                plsc.store_scatter(obufs[b], [pos], d0)
                plsc.store_scatter(obufs[b], [pos + 1], d1)
                plsc.store_scatter(obufs[b], [pos + 2], d2)
                plsc.store_scatter(obufs[b], [pos + 3], r2)
            pltpu.async_copy(obufs[b], o_h.at[pl.ds((base + j * CG3) * 4,
                                                    4 * CG3)], osems[b])
        return ()

    lax.fori_loop(0, CPT3 // NB3, grp_fn, ())
    for b in range(NB3):
        pltpu.make_async_copy(obufs[b], o_h.at[pl.ds(0, 4 * CG3)],
                              osems[b]).wait()


def _sc_drr(x_flat, row2, col2):
    f32 = jnp.float32
    fn = pl.kernel(
        _drr_body,
        out_type=jax.ShapeDtypeStruct((E * 4,), f32),
        mesh=plsc.VectorSubcoreMesh(core_axis_name="c", subcore_axis_name="s"),
        scratch_types=[pltpu.VMEM((4 * N,), f32),
                       pltpu.VMEM((E // 32,), jnp.int32),
                       pltpu.VMEM((E // 32,), jnp.int32)]
        + [pltpu.VMEM((4 * CG3,), f32)] * NB3
        + [pltpu.SemaphoreType.DMA] * NB3,
        compiler_params=pltpu.CompilerParams(needs_layout_passes=False),
    )
    return fn(x_flat, row2, col2)


def kernel(atom_feature, coords, edge_index, edge_attr, params):
    f32 = jnp.float32
    row = edge_index[0].astype(jnp.int32)
    col = edge_index[1].astype(jnp.int32)
    x40 = jnp.pad(coords.astype(f32), ((0, 0), (0, 1)))

    w_in, b_in = params['emb_in']
    wo, bo = params['emb_out']
    lps = params['layers']

    def split_l(lp):
        w1, b1 = lp['edge1']
        return dict(
            w1a=w1[:H], w1b=w1[H:2 * H], rvec=w1[2 * H:2 * H + 1],
            w1e=w1[2 * H + 1:], b1=b1.reshape(1, H),
            w2=lp['edge2'][0], b2=lp['edge2'][1].reshape(1, H),
            cw1=lp['coord1'][0], cb1=lp['coord1'][1].reshape(1, H),
            c2w=lp['coord2_w'].reshape(1, H),
            nw1a=lp['node1'][0][:H], nw1b=lp['node1'][0][H:],
            nb1=lp['node1'][1].reshape(1, H),
            nw2=lp['node2'][0], nb2=lp['node2'][1].reshape(1, H),
        )

    l1, l2 = split_l(lps[0]), split_l(lps[1])
    row3d = row.reshape(NS, CPT, CG)
    col3d = col.reshape(NS, CPT, CG)
    row4d = row.reshape(32, NGS, NBS, CGS)
    row2 = row.reshape(32, E // 32)
    col2 = col.reshape(32, E // 32)
    zer = jnp.zeros((CGS, H), f32)
    zer1 = jnp.zeros((4 * RPT,), f32)

    h, ta, tb = _node_pre(atom_feature, w_in, b_in.reshape(1, H),
                          l1['w1a'], l1['w1b'])
    ga, gb = _sc_gather(ta, tb, row3d, col3d)
    drr = _sc_drr(x40.reshape(4 * N), row2, col2).reshape(E, 4)
    m, aux = _edge1(ga, gb, drr, edge_attr, l1['w1e'], l1['rvec'], l1['b1'],
                    l1['w2'], l1['b2'], l1['cw1'], l1['cb1'], l1['c2w'])
    mp, oa = _sc_scatter_aux(m, aux.reshape(4 * E), row4d, row2, zer, zer1)
    ap = oa.reshape(2, NP, 4)
    h2, ta2, tb2, x41 = _node1(h, mp, ap, x40, l1['nw1a'], l1['nw1b'],
                               l1['nb1'], l1['nw2'], l1['nb2'],
                               l2['w1a'], l2['w1b'])

    ga2, gb2 = _sc_gather(ta2, tb2, row3d, col3d)
    drr2 = _sc_drr(x41.reshape(4 * N), row2, col2).reshape(E, 4)
    m2 = _edge2(ga2, gb2, drr2, edge_attr, l2['w1e'], l2['rvec'], l2['b1'],
                l2['w2'], l2['b2'])
    mp2 = _sc_scatter(m2, row4d, zer)
    return _node_out(h2, mp2, l2['nw1a'], l2['nw1b'], l2['nb1'],
                     l2['nw2'], l2['nb2'], wo, bo.reshape(1, H))


# speedup vs baseline: 5.1360x; 1.0080x over previous
import functools

import jax
import jax.numpy as jnp
from jax import lax
from jax.experimental import pallas as pl
from jax.experimental.pallas import tpu as pltpu
from jax.experimental.pallas import tpu_sc as plsc

N = 10000
E = 320000
H = 128
ED = 16

BN = 2000
BE = 8000


def _silu(x):
    return x * jax.nn.sigmoid(x)


def _full(shape):
    return pl.BlockSpec(shape, lambda i: (0,) * len(shape))


def _rows(bsz, width):
    return pl.BlockSpec((bsz, width), lambda i: (i, 0))


def _pre_body(h0, w_in, b_in, w1a, w1b, h_out, ta_out, tb_out):
    h = jnp.dot(h0[...], w_in[...], preferred_element_type=jnp.float32) + b_in[...]
    h_out[...] = h
    ta_out[...] = jnp.dot(h, w1a[...], preferred_element_type=jnp.float32)
    tb_out[...] = jnp.dot(h, w1b[...], preferred_element_type=jnp.float32)


def _node_pre(h0, w_in, b_in, w1a, w1b):
    f32 = jnp.float32
    return pl.pallas_call(
        _pre_body,
        grid=(N // BN,),
        in_specs=[_rows(BN, H), _full((H, H)), _full((1, H)), _full((H, H)),
                  _full((H, H))],
        out_specs=[_rows(BN, H), _rows(BN, H), _rows(BN, H)],
        out_shape=[jax.ShapeDtypeStruct((N, H), f32),
                   jax.ShapeDtypeStruct((N, H), f32),
                   jax.ShapeDtypeStruct((N, H), f32)],
    )(h0, w_in, b_in, w1a, w1b)


def _edge1_body(ga, gb, drr, ea, w1e, rvec, b1r, w2, b2, cw1, cb1, c2w,
                m_out, aux_out):
    dv = drr[...]
    lane = jax.lax.broadcasted_iota(jnp.int32, dv.shape, 1)
    diff = jnp.where(lane == 3, 0.0, dv)
    radial = dv[:, 3:4]
    pre = (ga[...] + gb[...]
           + jnp.dot(ea[...], w1e[...], preferred_element_type=jnp.float32)
           + radial * rvec[...] + b1r[...])
    m1 = _silu(pre)
    m = _silu(jnp.dot(m1, w2[...], preferred_element_type=jnp.float32) + b2[...])
    cm = _silu(jnp.dot(m, cw1[...], preferred_element_type=jnp.float32) + cb1[...])
    s = jnp.sum(cm * c2w[...], axis=1, keepdims=True)
    aux = jnp.where(lane == 3, 1.0, diff * s)
    m_out[...] = m
    aux_out[...] = aux


def _edge1(ga, gb, drr, ea, w1e, rvec, b1r, w2, b2, cw1, cb1, c2w):
    f32 = jnp.float32
    return pl.pallas_call(
        _edge1_body,
        grid=(E // BE,),
        in_specs=[_rows(BE, H), _rows(BE, H), _rows(BE, 4),
                  _rows(BE, ED), _full((ED, H)), _full((1, H)), _full((1, H)),
                  _full((H, H)), _full((1, H)), _full((H, H)), _full((1, H)),
                  _full((1, H))],
        out_specs=[_rows(BE, H), _rows(BE, 4)],
        out_shape=[jax.ShapeDtypeStruct((E, H), f32),
                   jax.ShapeDtypeStruct((E, 4), f32)],
    )(ga, gb, drr, ea, w1e, rvec, b1r, w2, b2, cw1, cb1, c2w)


def _edge2_body(ga, gb, drr, ea, w1e, rvec, b1r, w2, b2, m_out):
    radial = drr[...][:, 3:4]
    pre = (ga[...] + gb[...]
           + jnp.dot(ea[...], w1e[...], preferred_element_type=jnp.float32)
           + radial * rvec[...] + b1r[...])
    m1 = _silu(pre)
    m_out[...] = _silu(jnp.dot(m1, w2[...], preferred_element_type=jnp.float32)
                       + b2[...])


def _edge2(ga, gb, drr, ea, w1e, rvec, b1r, w2, b2):
    return pl.pallas_call(
        _edge2_body,
        grid=(E // BE,),
        in_specs=[_rows(BE, H), _rows(BE, H), _rows(BE, 4),
                  _rows(BE, ED), _full((ED, H)), _full((1, H)), _full((1, H)),
                  _full((H, H)), _full((1, H))],
        out_specs=_rows(BE, H),
        out_shape=jax.ShapeDtypeStruct((E, H), jnp.float32),
    )(ga, gb, drr, ea, w1e, rvec, b1r, w2, b2)


def _node1_body(h, mp0, mp1, ap0, ap1, x4, nw1a, nw1b, nb1, nw2, nb2,
                w1a, w1b, h_out, ta_out, tb_out, x_out):
    hh = h[...]
    magg = mp0[0] + mp1[0]
    av = ap0[0] + ap1[0]
    cnt = jnp.clip(av[:, 3:4], 1.0, None)
    lane = jax.lax.broadcasted_iota(jnp.int32, av.shape, 1)
    x_out[...] = x4[...] + jnp.where(lane < 3, av / cnt, 0.0)
    nh = _silu(jnp.dot(hh, nw1a[...], preferred_element_type=jnp.float32)
               + jnp.dot(magg, nw1b[...], preferred_element_type=jnp.float32)
               + nb1[...])
    h_new = hh + jnp.dot(nh, nw2[...], preferred_element_type=jnp.float32) + nb2[...]
    h_out[...] = h_new
    ta_out[...] = jnp.dot(h_new, w1a[...], preferred_element_type=jnp.float32)
    tb_out[...] = jnp.dot(h_new, w1b[...], preferred_element_type=jnp.float32)


def _part(bsz, width, p):
    return pl.BlockSpec((1, bsz, width), lambda i, _p=p: (_p, i, 0))


def _node1(h, mp, ap, x4, nw1a, nw1b, nb1, nw2, nb2, w1a, w1b):
    f32 = jnp.float32
    return pl.pallas_call(
        _node1_body,
        grid=(N // BN,),
        in_specs=[_rows(BN, H), _part(BN, H, 0), _part(BN, H, 1),
                  _part(BN, 4, 0), _part(BN, 4, 1), _rows(BN, 4),
                  _full((H, H)), _full((H, H)), _full((1, H)), _full((H, H)),
                  _full((1, H)), _full((H, H)), _full((H, H))],
        out_specs=[_rows(BN, H), _rows(BN, H), _rows(BN, H), _rows(BN, 4)],
        out_shape=[jax.ShapeDtypeStruct((N, H), f32),
                   jax.ShapeDtypeStruct((N, H), f32),
                   jax.ShapeDtypeStruct((N, H), f32),
                   jax.ShapeDtypeStruct((N, 4), f32)],
    )(h, mp, mp, ap, ap, x4, nw1a, nw1b, nb1, nw2, nb2, w1a, w1b)


def _nodeout_body(h, mp0, mp1, nw1a, nw1b, nb1, nw2, nb2, wo, bo, out):
    hh = h[...]
    magg = mp0[0] + mp1[0]
    nh = _silu(jnp.dot(hh, nw1a[...], preferred_element_type=jnp.float32)
               + jnp.dot(magg, nw1b[...], preferred_element_type=jnp.float32)
               + nb1[...])
    h_new = hh + jnp.dot(nh, nw2[...], preferred_element_type=jnp.float32) + nb2[...]
    out[...] = jnp.dot(h_new, wo[...], preferred_element_type=jnp.float32) + bo[...]


def _node_out(h, mp, nw1a, nw1b, nb1, nw2, nb2, wo, bo):
    return pl.pallas_call(
        _nodeout_body,
        grid=(N // BN,),
        in_specs=[_rows(BN, H), _part(BN, H, 0), _part(BN, H, 1),
                  _full((H, H)), _full((H, H)),
                  _full((1, H)), _full((H, H)), _full((1, H)), _full((H, H)),
                  _full((1, H))],
        out_specs=_rows(BN, H),
        out_shape=jax.ShapeDtypeStruct((N, H), jnp.float32),
    )(h, mp, mp, nw1a, nw1b, nb1, nw2, nb2, wo, bo)


NS = 16
EPT = E // NS
CG = 80
NB = 5
CPT = EPT // CG
NG = CPT // NB


def _gather_body(ta_h, tb_h, r_h, c_h, ga_h, gb_h, idxv, buf, *sems):
    gsems, ssems = sems[:NB], sems[NB:]
    cid = lax.axis_index("c")
    sid = lax.axis_index("s")

    def run(tab, ih, oh):
        base = sid * EPT
        pltpu.sync_copy(ih.at[sid], idxv)

        def grp(g, _):
            handles = []
            for b in range(NB):
                j = g * NB + b

                @pl.when(g > 0)
                def _():
                    pltpu.make_async_copy(buf.at[b], oh.at[pl.ds(0, CG)],
                                          ssems[b]).wait()

                handles.append(pltpu.async_copy(tab.at[idxv.at[j]], buf.at[b],
                                                gsems[b]))
            for b in range(NB):
                off = base + (g * NB + b) * CG
                handles[b].wait()
                pltpu.async_copy(buf.at[b], oh.at[pl.ds(off, CG)], ssems[b])
            return ()

        lax.fori_loop(0, NG, grp, ())
        for b in range(NB):
            pltpu.make_async_copy(buf.at[b], oh.at[pl.ds(0, CG)],
                                  ssems[b]).wait()

    @pl.when(cid == 0)
    def _():
        run(ta_h, r_h, ga_h)

    @pl.when(cid == 1)
    def _():
        run(tb_h, c_h, gb_h)


def _sc_gather(ta, tb, row3d, col3d):
    f32 = jnp.float32
    fn = pl.kernel(
        _gather_body,
        out_type=[jax.ShapeDtypeStruct((E, H), f32),
                  jax.ShapeDtypeStruct((E, H), f32)],
        mesh=plsc.VectorSubcoreMesh(core_axis_name="c", subcore_axis_name="s"),
        scratch_types=[pltpu.VMEM((CPT, CG), jnp.int32),
                       pltpu.VMEM((NB, CG, H), f32)]
        + [pltpu.SemaphoreType.DMA] * (2 * NB),
    )
    return fn(ta, tb, row3d, col3d)


NP = 10240
RPT = NP // NS
CGS = 40
NBS = 5
CPTS = (E // 32) // CGS


NGS = CPTS // NBS


def _scatter_body(m_h, r_h, z_h, om_h, idxg, mring, accm, *sems):
    asems = sems[:NBS]
    zsem, isem = sems[NBS], sems[NBS + 1]
    cid = lax.axis_index("c")
    sid = lax.axis_index("s")

    pltpu.sync_copy(z_h, mring.at[0])
    for i in range(RPT // CGS):
        pltpu.async_copy(mring.at[0], accm.at[pl.ds(sid * RPT + i * CGS, CGS)],
                         zsem)
    for i in range(RPT // CGS):
        pltpu.make_async_copy(mring.at[0], accm.at[pl.ds(0, CGS)], zsem).wait()
    plsc.subcore_barrier()

    tid = cid * NS + sid
    base = tid * (E // 32)
    r_t = r_h.at[tid]
    pltpu.sync_copy(r_t.at[0], idxg.at[0])

    def grp(g, _):
        pb = lax.rem(g, 2)

        @pl.when(g > 0)
        def _():
            pltpu.make_async_copy(r_t.at[0], idxg.at[0], isem).wait()

        handles = []
        for b in range(NBS):
            off = base + (g * NBS + b) * CGS

            @pl.when(g > 0)
            def _():
                pltpu.make_async_copy(mring.at[b], accm.at[pl.ds(0, CGS)],
                                      asems[b]).wait()

            handles.append(pltpu.async_copy(m_h.at[pl.ds(off, CGS)],
                                            mring.at[b], asems[b]))
        for b in range(NBS):
            handles[b].wait()
            pltpu.async_copy(mring.at[b], accm.at[idxg.at[pb, b]],
                             asems[b], add=True)

        @pl.when(g + 1 < NGS)
        def _():
            pltpu.async_copy(r_t.at[g + 1], idxg.at[1 - pb], isem)

        return ()

    lax.fori_loop(0, NGS, grp, ())
    for b in range(NBS):
        pltpu.make_async_copy(mring.at[b], accm.at[pl.ds(0, CGS)],
                              asems[b]).wait()
    plsc.subcore_barrier()

    @pl.when(sid < NS - 1)
    def _():
        pltpu.sync_copy(accm.at[pl.ds(sid * RPT, RPT)],
                        om_h.at[cid].at[pl.ds(sid * RPT, RPT)])

    @pl.when(sid == NS - 1)
    def _():
        pltpu.sync_copy(accm.at[pl.ds((NS - 1) * RPT, 400)],
                        om_h.at[cid].at[pl.ds((NS - 1) * RPT, 400)])


def _sc_scatter(m, row4d, zer):
    f32 = jnp.float32
    fn = pl.kernel(
        _scatter_body,
        out_type=jax.ShapeDtypeStruct((2, N, H), f32),
        mesh=plsc.VectorSubcoreMesh(core_axis_name="c", subcore_axis_name="s"),
        scratch_types=[pltpu.VMEM((2, NBS, CGS), jnp.int32),
                       pltpu.VMEM((NBS, CGS, H), f32),
                       pltpu.VMEM_SHARED((NP, H), f32)]
        + [pltpu.SemaphoreType.DMA] * (NBS + 2),
    )
    return fn(m, row4d, zer)


def _scatter_aux_body(m_h, a_h, r_h, r2_h, z_h, z1_h, om_h, oa_h,
                      idxg, idxf, mring, accm, acc4, zb4, *rest):
    vbufs = rest[:NBS]
    eb0 = rest[NBS:2 * NBS]
    eb1 = rest[2 * NBS:3 * NBS]
    sems = rest[3 * NBS:]
    asems = sems[:NBS]
    zsem, isem = sems[NBS], sems[NBS + 1]
    cid = lax.axis_index("c")
    sid = lax.axis_index("s")

    pltpu.sync_copy(z_h, mring.at[0])
    pltpu.sync_copy(z1_h, zb4)
    for i in range(RPT // CGS):
        pltpu.async_copy(mring.at[0], accm.at[pl.ds(sid * RPT + i * CGS, CGS)],
                         zsem)
    pltpu.async_copy(zb4, acc4.at[pl.ds(sid * 4 * RPT, 4 * RPT)], zsem)
    for i in range(RPT // CGS):
        pltpu.make_async_copy(mring.at[0], accm.at[pl.ds(0, CGS)], zsem).wait()
    pltpu.make_async_copy(zb4, acc4.at[pl.ds(0, 4 * RPT)], zsem).wait()
    plsc.subcore_barrier()

    tid = cid * NS + sid
    base = tid * (E // 32)
    r_t = r_h.at[tid]
    pltpu.sync_copy(r_t.at[0], idxg.at[0])
    pltpu.sync_copy(r2_h.at[tid], idxf)

    def grp(g, _):
        pb = lax.rem(g, 2)

        @pl.when(g > 0)
        def _():
            pltpu.make_async_copy(r_t.at[0], idxg.at[0], isem).wait()

        handles = []
        for b in range(NBS):
            off = base + (g * NBS + b) * CGS

            @pl.when(g > 0)
            def _():
                pltpu.make_async_copy(mring.at[b], accm.at[pl.ds(0, CGS)],
                                      asems[b]).wait()
                pltpu.make_async_copy(vbufs[b].at[pl.ds(0, 2 * CGS)],
                                      acc4.at[pl.ds(0, 2 * CGS)],
                                      asems[b]).wait()
                pltpu.make_async_copy(vbufs[b].at[pl.ds(0, 2 * CGS)],
                                      acc4.at[pl.ds(0, 2 * CGS)],
                                      asems[b]).wait()

            handles.append(pltpu.async_copy(m_h.at[pl.ds(off, CGS)],
                                            mring.at[b], asems[b]))
            handles.append(pltpu.async_copy(a_h.at[pl.ds(off * 4, 4 * CGS)],
                                            vbufs[b], asems[b]))
        for b in range(NBS):
            j = g * NBS + b
            handles[2 * b].wait()
            handles[2 * b + 1].wait()
            for g8 in range(4 * CGS // 16):
                lanes = lax.iota(jnp.int32, 16)
                e = j * CGS + g8 * 4 + lanes // 4
                rv = plsc.load_gather(idxf, [e])
                el = rv * 4 + lax.rem(lanes, 4)
                if g8 < 2 * CGS // 16:
                    plsc.store_scatter(eb0[b], [g8 * 16 + lanes], el)
                else:
                    plsc.store_scatter(eb1[b],
                                       [(g8 - 2 * CGS // 16) * 16 + lanes], el)
            pltpu.async_copy(mring.at[b], accm.at[idxg.at[pb, b]],
                             asems[b], add=True)
            pltpu.async_copy(vbufs[b].at[pl.ds(0, 2 * CGS)], acc4.at[eb0[b]],
                             asems[b], add=True)
            pltpu.async_copy(vbufs[b].at[pl.ds(2 * CGS, 2 * CGS)],
                             acc4.at[eb1[b]], asems[b], add=True)

        @pl.when(g + 1 < NGS)
        def _():
            pltpu.async_copy(r_t.at[g + 1], idxg.at[1 - pb], isem)

        return ()

    lax.fori_loop(0, NGS, grp, ())
    for b in range(NBS):
        pltpu.make_async_copy(mring.at[b], accm.at[pl.ds(0, CGS)],
                              asems[b]).wait()
        pltpu.make_async_copy(vbufs[b].at[pl.ds(0, 2 * CGS)],
                              acc4.at[pl.ds(0, 2 * CGS)], asems[b]).wait()
        pltpu.make_async_copy(vbufs[b].at[pl.ds(0, 2 * CGS)],
                              acc4.at[pl.ds(0, 2 * CGS)], asems[b]).wait()
    plsc.subcore_barrier()

    pltpu.sync_copy(acc4.at[pl.ds(sid * 4 * RPT, 4 * RPT)],
                    oa_h.at[cid].at[pl.ds(sid * 4 * RPT, 4 * RPT)])

    @pl.when(sid < NS - 1)
    def _():
        pltpu.sync_copy(accm.at[pl.ds(sid * RPT, RPT)],
                        om_h.at[cid].at[pl.ds(sid * RPT, RPT)])

    @pl.when(sid == NS - 1)
    def _():
        pltpu.sync_copy(accm.at[pl.ds((NS - 1) * RPT, 400)],
                        om_h.at[cid].at[pl.ds((NS - 1) * RPT, 400)])


def _sc_scatter_aux(m, aux_flat, row4d, row2, zer, zer1):
    f32 = jnp.float32
    fn = pl.kernel(
        _scatter_aux_body,
        out_type=[jax.ShapeDtypeStruct((2, N, H), f32),
                  jax.ShapeDtypeStruct((2, 4 * NP), f32)],
        mesh=plsc.VectorSubcoreMesh(core_axis_name="c", subcore_axis_name="s"),
        scratch_types=[pltpu.VMEM((2, NBS, CGS), jnp.int32),
                       pltpu.VMEM((E // 32,), jnp.int32),
                       pltpu.VMEM((NBS, CGS, H), f32),
                       pltpu.VMEM_SHARED((NP, H), f32),
                       pltpu.VMEM_SHARED((4 * NP,), f32),
                       pltpu.VMEM((4 * RPT,), f32)]
        + [pltpu.VMEM((4 * CGS,), f32)] * NBS
        + [pltpu.VMEM((2 * CGS,), jnp.int32)] * (2 * NBS)
        + [pltpu.SemaphoreType.DMA] * (NBS + 2),
        compiler_params=pltpu.CompilerParams(needs_layout_passes=False),
    )
    return fn(m, aux_flat, row4d, row2, zer, zer1)


NB3 = 5
CG3 = 80
CPT3 = (E // 32) // CG3


def _drr_body(x_h, r_h, c_h, o_h, xv, ridx, cidx, *rest):
    obufs, osems = rest[:NB3], rest[NB3:]
    cid = lax.axis_index("c")
    sid = lax.axis_index("s")
    tid = cid * NS + sid
    base = tid * (E // 32)
    pltpu.sync_copy(x_h, xv)
    pltpu.sync_copy(r_h.at[tid], ridx)
    pltpu.sync_copy(c_h.at[tid], cidx)
    lanes = lax.iota(jnp.int32, 16)

    def grp_fn(g, _):
        for b in range(NB3):
            j = g * NB3 + b

            @pl.when(g > 0)
            def _():
                pltpu.make_async_copy(obufs[b], o_h.at[pl.ds(0, 4 * CG3)],
                                      osems[b]).wait()

            for grp in range(CG3 // 16):
                ev = j * CG3 + grp * 16 + lanes
                rid = plsc.load_gather(ridx, [ev])
                cidv = plsc.load_gather(cidx, [ev])
                d0 = (plsc.load_gather(xv, [rid * 4])
                      - plsc.load_gather(xv, [cidv * 4]))
                d1 = (plsc.load_gather(xv, [rid * 4 + 1])
                      - plsc.load_gather(xv, [cidv * 4 + 1]))
                d2 = (plsc.load_gather(xv, [rid * 4 + 2])
                      - plsc.load_gather(xv, [cidv * 4 + 2]))
                r2 = d0 * d0 + d1 * d1 + d2 * d2
                pos = (grp * 16 + lanes) * 4
                plsc.store_scatter(obufs[b], [pos], d0)
                plsc.store_scatter(obufs[b], [pos + 1], d1)
                plsc.store_scatter(obufs[b], [pos + 2], d2)
                plsc.store_scatter(obufs[b], [pos + 3], r2)
            pltpu.async_copy(obufs[b], o_h.at[pl.ds((base + j * CG3) * 4,
                                                    4 * CG3)], osems[b])
        return ()

    lax.fori_loop(0, CPT3 // NB3, grp_fn, ())
    for b in range(NB3):
        pltpu.make_async_copy(obufs[b], o_h.at[pl.ds(0, 4 * CG3)],
                              osems[b]).wait()


def _sc_drr(x_flat, row2, col2):
    f32 = jnp.float32
    fn = pl.kernel(
        _drr_body,
        out_type=jax.ShapeDtypeStruct((E * 4,), f32),
        mesh=plsc.VectorSubcoreMesh(core_axis_name="c", subcore_axis_name="s"),
        scratch_types=[pltpu.VMEM((4 * N,), f32),
                       pltpu.VMEM((E // 32,), jnp.int32),
                       pltpu.VMEM((E // 32,), jnp.int32)]
        + [pltpu.VMEM((4 * CG3,), f32)] * NB3
        + [pltpu.SemaphoreType.DMA] * NB3,
        compiler_params=pltpu.CompilerParams(needs_layout_passes=False),
    )
    return fn(x_flat, row2, col2)


def kernel(atom_feature, coords, edge_index, edge_attr, params):
    f32 = jnp.float32
    row = edge_index[0].astype(jnp.int32)
    col = edge_index[1].astype(jnp.int32)
    x40 = jnp.pad(coords.astype(f32), ((0, 0), (0, 1)))

    w_in, b_in = params['emb_in']
    wo, bo = params['emb_out']
    lps = params['layers']

    def split_l(lp):
        w1, b1 = lp['edge1']
        return dict(
            w1a=w1[:H], w1b=w1[H:2 * H], rvec=w1[2 * H:2 * H + 1],
            w1e=w1[2 * H + 1:], b1=b1.reshape(1, H),
            w2=lp['edge2'][0], b2=lp['edge2'][1].reshape(1, H),
            cw1=lp['coord1'][0], cb1=lp['coord1'][1].reshape(1, H),
            c2w=lp['coord2_w'].reshape(1, H),
            nw1a=lp['node1'][0][:H], nw1b=lp['node1'][0][H:],
            nb1=lp['node1'][1].reshape(1, H),
            nw2=lp['node2'][0], nb2=lp['node2'][1].reshape(1, H),
        )

    l1, l2 = split_l(lps[0]), split_l(lps[1])
    row3d = row.reshape(NS, CPT, CG)
    col3d = col.reshape(NS, CPT, CG)
    row4d = row.reshape(32, NGS, NBS, CGS)
    row2 = row.reshape(32, E // 32)
    col2 = col.reshape(32, E // 32)
    zer = jnp.zeros((CGS, H), f32)
    zer1 = jnp.zeros((4 * RPT,), f32)

    h, ta, tb = _node_pre(atom_feature, w_in, b_in.reshape(1, H),
                          l1['w1a'], l1['w1b'])
    ga, gb = _sc_gather(ta, tb, row3d, col3d)
    drr = _sc_drr(x40.reshape(4 * N), row2, col2).reshape(E, 4)
    m, aux = _edge1(ga, gb, drr, edge_attr, l1['w1e'], l1['rvec'], l1['b1'],
                    l1['w2'], l1['b2'], l1['cw1'], l1['cb1'], l1['c2w'])
    mp, oa = _sc_scatter_aux(m, aux.reshape(4 * E), row4d, row2, zer, zer1)
    ap = oa.reshape(2, NP, 4)
    h2, ta2, tb2, x41 = _node1(h, mp, ap, x40, l1['nw1a'], l1['nw1b'],
                               l1['nb1'], l1['nw2'], l1['nb2'],
                               l2['w1a'], l2['w1b'])

    ga2, gb2 = _sc_gather(ta2, tb2, row3d, col3d)
    drr2 = _sc_drr(x41.reshape(4 * N), row2, col2).reshape(E, 4)
    m2 = _edge2(ga2, gb2, drr2, edge_attr, l2['w1e'], l2['rvec'], l2['b1'],
                l2['w2'], l2['b2'])
    mp2 = _sc_scatter(m2, row4d, zer)
    return _node_out(h2, mp2, l2['nw1a'], l2['nw1b'], l2['nb1'],
                     l2['nw2'], l2['nb2'], wo, bo.reshape(1, H))


# speedup vs baseline: 5.6399x; 1.0981x over previous
import functools

import jax
import jax.numpy as jnp
from jax import lax
from jax.experimental import pallas as pl
from jax.experimental.pallas import tpu as pltpu
from jax.experimental.pallas import tpu_sc as plsc

N = 10000
E = 320000
H = 128
ED = 16

BN = 2000
BE = 8000


def _silu(x):
    return x * jax.nn.sigmoid(x)


def _full(shape):
    return pl.BlockSpec(shape, lambda i: (0,) * len(shape))


def _rows(bsz, width):
    return pl.BlockSpec((bsz, width), lambda i: (i, 0))


def _pre_body(h0, w_in, b_in, w1a, w1b, h_out, ta_out, tb_out):
    h = jnp.dot(h0[...], w_in[...], preferred_element_type=jnp.float32) + b_in[...]
    h_out[...] = h
    ta_out[...] = jnp.dot(h, w1a[...], preferred_element_type=jnp.float32)
    tb_out[...] = jnp.dot(h, w1b[...], preferred_element_type=jnp.float32)


def _node_pre(h0, w_in, b_in, w1a, w1b):
    f32 = jnp.float32
    return pl.pallas_call(
        _pre_body,
        grid=(N // BN,),
        in_specs=[_rows(BN, H), _full((H, H)), _full((1, H)), _full((H, H)),
                  _full((H, H))],
        out_specs=[_rows(BN, H), _rows(BN, H), _rows(BN, H)],
        out_shape=[jax.ShapeDtypeStruct((N, H), f32),
                   jax.ShapeDtypeStruct((N, H), f32),
                   jax.ShapeDtypeStruct((N, H), f32)],
    )(h0, w_in, b_in, w1a, w1b)


def _edge1_body(ga, gb, drr, ea, w1e, rvec, b1r, w2, b2, cw1, cb1, c2w,
                m_out, aux_out):
    dv = drr[...]
    lane = jax.lax.broadcasted_iota(jnp.int32, dv.shape, 1)
    diff = jnp.where(lane == 3, 0.0, dv)
    radial = dv[:, 3:4]
    pre = (ga[...] + gb[...]
           + jnp.dot(ea[...], w1e[...], preferred_element_type=jnp.float32)
           + radial * rvec[...] + b1r[...])
    m1 = _silu(pre)
    m = _silu(jnp.dot(m1, w2[...], preferred_element_type=jnp.float32) + b2[...])
    cm = _silu(jnp.dot(m, cw1[...], preferred_element_type=jnp.float32) + cb1[...])
    s = jnp.sum(cm * c2w[...], axis=1, keepdims=True)
    aux = jnp.where(lane == 3, 1.0, diff * s)
    m_out[...] = m
    aux_out[...] = aux


def _edge1(ga, gb, drr, ea, w1e, rvec, b1r, w2, b2, cw1, cb1, c2w):
    f32 = jnp.float32
    return pl.pallas_call(
        _edge1_body,
        grid=(E // BE,),
        in_specs=[_rows(BE, H), _rows(BE, H), _rows(BE, 4),
                  _rows(BE, ED), _full((ED, H)), _full((1, H)), _full((1, H)),
                  _full((H, H)), _full((1, H)), _full((H, H)), _full((1, H)),
                  _full((1, H))],
        out_specs=[_rows(BE, H), _rows(BE, 4)],
        out_shape=[jax.ShapeDtypeStruct((E, H), f32),
                   jax.ShapeDtypeStruct((E, 4), f32)],
    )(ga, gb, drr, ea, w1e, rvec, b1r, w2, b2, cw1, cb1, c2w)


def _edge2_body(ga, gb, drr, ea, w1e, rvec, b1r, w2, b2, m_out):
    radial = drr[...][:, 3:4]
    pre = (ga[...] + gb[...]
           + jnp.dot(ea[...], w1e[...], preferred_element_type=jnp.float32)
           + radial * rvec[...] + b1r[...])
    m1 = _silu(pre)
    m_out[...] = _silu(jnp.dot(m1, w2[...], preferred_element_type=jnp.float32)
                       + b2[...])


def _edge2(ga, gb, drr, ea, w1e, rvec, b1r, w2, b2):
    return pl.pallas_call(
        _edge2_body,
        grid=(E // BE,),
        in_specs=[_rows(BE, H), _rows(BE, H), _rows(BE, 4),
                  _rows(BE, ED), _full((ED, H)), _full((1, H)), _full((1, H)),
                  _full((H, H)), _full((1, H))],
        out_specs=_rows(BE, H),
        out_shape=jax.ShapeDtypeStruct((E, H), jnp.float32),
    )(ga, gb, drr, ea, w1e, rvec, b1r, w2, b2)


def _node1_body(h, mp0, mp1, ap0, ap1, x4, nw1a, nw1b, nb1, nw2, nb2,
                w1a, w1b, h_out, ta_out, tb_out, x_out):
    hh = h[...]
    magg = mp0[0] + mp1[0]
    av = ap0[0] + ap1[0]
    cnt = jnp.clip(av[:, 3:4], 1.0, None)
    lane = jax.lax.broadcasted_iota(jnp.int32, av.shape, 1)
    x_out[...] = x4[...] + jnp.where(lane < 3, av / cnt, 0.0)
    nh = _silu(jnp.dot(hh, nw1a[...], preferred_element_type=jnp.float32)
               + jnp.dot(magg, nw1b[...], preferred_element_type=jnp.float32)
               + nb1[...])
    h_new = hh + jnp.dot(nh, nw2[...], preferred_element_type=jnp.float32) + nb2[...]
    h_out[...] = h_new
    ta_out[...] = jnp.dot(h_new, w1a[...], preferred_element_type=jnp.float32)
    tb_out[...] = jnp.dot(h_new, w1b[...], preferred_element_type=jnp.float32)


def _part(bsz, width, p):
    return pl.BlockSpec((1, bsz, width), lambda i, _p=p: (_p, i, 0))


def _node1(h, mp, ap, x4, nw1a, nw1b, nb1, nw2, nb2, w1a, w1b):
    f32 = jnp.float32
    return pl.pallas_call(
        _node1_body,
        grid=(N // BN,),
        in_specs=[_rows(BN, H), _part(BN, H, 0), _part(BN, H, 1),
                  _part(BN, 4, 0), _part(BN, 4, 1), _rows(BN, 4),
                  _full((H, H)), _full((H, H)), _full((1, H)), _full((H, H)),
                  _full((1, H)), _full((H, H)), _full((H, H))],
        out_specs=[_rows(BN, H), _rows(BN, H), _rows(BN, H), _rows(BN, 4)],
        out_shape=[jax.ShapeDtypeStruct((N, H), f32),
                   jax.ShapeDtypeStruct((N, H), f32),
                   jax.ShapeDtypeStruct((N, H), f32),
                   jax.ShapeDtypeStruct((N, 4), f32)],
    )(h, mp, mp, ap, ap, x4, nw1a, nw1b, nb1, nw2, nb2, w1a, w1b)


def _nodeout_body(h, mp0, mp1, nw1a, nw1b, nb1, nw2, nb2, wo, bo, out):
    hh = h[...]
    magg = mp0[0] + mp1[0]
    nh = _silu(jnp.dot(hh, nw1a[...], preferred_element_type=jnp.float32)
               + jnp.dot(magg, nw1b[...], preferred_element_type=jnp.float32)
               + nb1[...])
    h_new = hh + jnp.dot(nh, nw2[...], preferred_element_type=jnp.float32) + nb2[...]
    out[...] = jnp.dot(h_new, wo[...], preferred_element_type=jnp.float32) + bo[...]


def _node_out(h, mp, nw1a, nw1b, nb1, nw2, nb2, wo, bo):
    return pl.pallas_call(
        _nodeout_body,
        grid=(N // BN,),
        in_specs=[_rows(BN, H), _part(BN, H, 0), _part(BN, H, 1),
                  _full((H, H)), _full((H, H)),
                  _full((1, H)), _full((H, H)), _full((1, H)), _full((H, H)),
                  _full((1, H))],
        out_specs=_rows(BN, H),
        out_shape=jax.ShapeDtypeStruct((N, H), jnp.float32),
    )(h, mp, mp, nw1a, nw1b, nb1, nw2, nb2, wo, bo)


NS = 16
EPT = E // NS
CG = 80
NB = 5
CPT = EPT // CG
NG = CPT // NB


CGG = 40
NBG = 5
NGG = EPT // (CGG * NBG)
TRS = 640


def _gather_body(ta_h, tb_h, r_h, c_h, ga_h, gb_h, idxg, ring, tabS, *sems):
    asems = sems[:NBG]
    isem = sems[NBG]
    cid = lax.axis_index("c")
    sid = lax.axis_index("s")

    def run(tab, ih, oh):
        @pl.when(sid < NS - 1)
        def _():
            pltpu.sync_copy(tab.at[pl.ds(sid * TRS, TRS)],
                            tabS.at[pl.ds(sid * TRS, TRS)])

        @pl.when(sid == NS - 1)
        def _():
            pltpu.sync_copy(tab.at[pl.ds((NS - 1) * TRS, N - (NS - 1) * TRS)],
                            tabS.at[pl.ds((NS - 1) * TRS,
                                          N - (NS - 1) * TRS)])

        plsc.subcore_barrier()
        base = sid * EPT
        i_t = ih.at[sid]
        pltpu.sync_copy(i_t.at[0], idxg.at[0])

        def grp(g, _):
            pb = lax.rem(g, 2)

            @pl.when(g > 0)
            def _():
                pltpu.make_async_copy(i_t.at[0], idxg.at[0], isem).wait()

            handles = []
            for b in range(NBG):
                @pl.when(g > 0)
                def _():
                    pltpu.make_async_copy(ring.at[b], oh.at[pl.ds(0, CGG)],
                                          asems[b]).wait()

                handles.append(pltpu.async_copy(tabS.at[idxg.at[pb, b]],
                                                ring.at[b], asems[b]))
            for b in range(NBG):
                off = base + (g * NBG + b) * CGG
                handles[b].wait()
                pltpu.async_copy(ring.at[b], oh.at[pl.ds(off, CGG)], asems[b])

            @pl.when(g + 1 < NGG)
            def _():
                pltpu.async_copy(i_t.at[g + 1], idxg.at[1 - pb], isem)

            return ()

        lax.fori_loop(0, NGG, grp, ())
        for b in range(NBG):
            pltpu.make_async_copy(ring.at[b], oh.at[pl.ds(0, CGG)],
                                  asems[b]).wait()

    @pl.when(cid == 0)
    def _():
        run(ta_h, r_h, ga_h)

    @pl.when(cid == 1)
    def _():
        run(tb_h, c_h, gb_h)


def _sc_gather(ta, tb, row3d, col3d):
    f32 = jnp.float32
    fn = pl.kernel(
        _gather_body,
        out_type=[jax.ShapeDtypeStruct((E, H), f32),
                  jax.ShapeDtypeStruct((E, H), f32)],
        mesh=plsc.VectorSubcoreMesh(core_axis_name="c", subcore_axis_name="s"),
        scratch_types=[pltpu.VMEM((2, NBG, CGG), jnp.int32),
                       pltpu.VMEM((NBG, CGG, H), f32),
                       pltpu.VMEM_SHARED((N, H), f32)]
        + [pltpu.SemaphoreType.DMA] * (NBG + 1),
    )
    return fn(ta, tb, row3d, col3d)


NP = 10240
RPT = NP // NS
CGS = 40
NBS = 5
CPTS = (E // 32) // CGS


NGS = CPTS // NBS


def _scatter_body(m_h, r_h, z_h, om_h, idxg, mring, accm, *sems):
    asems = sems[:NBS]
    zsem, isem = sems[NBS], sems[NBS + 1]
    cid = lax.axis_index("c")
    sid = lax.axis_index("s")

    pltpu.sync_copy(z_h, mring.at[0])
    for i in range(RPT // CGS):
        pltpu.async_copy(mring.at[0], accm.at[pl.ds(sid * RPT + i * CGS, CGS)],
                         zsem)
    for i in range(RPT // CGS):
        pltpu.make_async_copy(mring.at[0], accm.at[pl.ds(0, CGS)], zsem).wait()
    plsc.subcore_barrier()

    tid = cid * NS + sid
    base = tid * (E // 32)
    r_t = r_h.at[tid]
    pltpu.sync_copy(r_t.at[0], idxg.at[0])

    def grp(g, _):
        pb = lax.rem(g, 2)

        @pl.when(g > 0)
        def _():
            pltpu.make_async_copy(r_t.at[0], idxg.at[0], isem).wait()

        handles = []
        for b in range(NBS):
            off = base + (g * NBS + b) * CGS

            @pl.when(g > 0)
            def _():
                pltpu.make_async_copy(mring.at[b], accm.at[pl.ds(0, CGS)],
                                      asems[b]).wait()

            handles.append(pltpu.async_copy(m_h.at[pl.ds(off, CGS)],
                                            mring.at[b], asems[b]))
        for b in range(NBS):
            handles[b].wait()
            pltpu.async_copy(mring.at[b], accm.at[idxg.at[pb, b]],
                             asems[b], add=True)

        @pl.when(g + 1 < NGS)
        def _():
            pltpu.async_copy(r_t.at[g + 1], idxg.at[1 - pb], isem)

        return ()

    lax.fori_loop(0, NGS, grp, ())
    for b in range(NBS):
        pltpu.make_async_copy(mring.at[b], accm.at[pl.ds(0, CGS)],
                              asems[b]).wait()
    plsc.subcore_barrier()

    @pl.when(sid < NS - 1)
    def _():
        pltpu.sync_copy(accm.at[pl.ds(sid * RPT, RPT)],
                        om_h.at[cid].at[pl.ds(sid * RPT, RPT)])

    @pl.when(sid == NS - 1)
    def _():
        pltpu.sync_copy(accm.at[pl.ds((NS - 1) * RPT, 400)],
                        om_h.at[cid].at[pl.ds((NS - 1) * RPT, 400)])


def _sc_scatter(m, row4d, zer):
    f32 = jnp.float32
    fn = pl.kernel(
        _scatter_body,
        out_type=jax.ShapeDtypeStruct((2, N, H), f32),
        mesh=plsc.VectorSubcoreMesh(core_axis_name="c", subcore_axis_name="s"),
        scratch_types=[pltpu.VMEM((2, NBS, CGS), jnp.int32),
                       pltpu.VMEM((NBS, CGS, H), f32),
                       pltpu.VMEM_SHARED((NP, H), f32)]
        + [pltpu.SemaphoreType.DMA] * (NBS + 2),
    )
    return fn(m, row4d, zer)


def _scatter_aux_body(m_h, a_h, r_h, r2_h, z_h, z1_h, om_h, oa_h,
                      idxg, idxf, mring, accm, acc4, zb4, *rest):
    vbufs = rest[:NBS]
    eb0 = rest[NBS:2 * NBS]
    eb1 = rest[2 * NBS:3 * NBS]
    sems = rest[3 * NBS:]
    asems = sems[:NBS]
    zsem, isem = sems[NBS], sems[NBS + 1]
    cid = lax.axis_index("c")
    sid = lax.axis_index("s")

    pltpu.sync_copy(z_h, mring.at[0])
    pltpu.sync_copy(z1_h, zb4)
    for i in range(RPT // CGS):
        pltpu.async_copy(mring.at[0], accm.at[pl.ds(sid * RPT + i * CGS, CGS)],
                         zsem)
    pltpu.async_copy(zb4, acc4.at[pl.ds(sid * 4 * RPT, 4 * RPT)], zsem)
    for i in range(RPT // CGS):
        pltpu.make_async_copy(mring.at[0], accm.at[pl.ds(0, CGS)], zsem).wait()
    pltpu.make_async_copy(zb4, acc4.at[pl.ds(0, 4 * RPT)], zsem).wait()
    plsc.subcore_barrier()

    tid = cid * NS + sid
    base = tid * (E // 32)
    r_t = r_h.at[tid]
    pltpu.sync_copy(r_t.at[0], idxg.at[0])
    pltpu.sync_copy(r2_h.at[tid], idxf)

    def grp(g, _):
        pb = lax.rem(g, 2)

        @pl.when(g > 0)
        def _():
            pltpu.make_async_copy(r_t.at[0], idxg.at[0], isem).wait()

        handles = []
        for b in range(NBS):
            off = base + (g * NBS + b) * CGS

            @pl.when(g > 0)
            def _():
                pltpu.make_async_copy(mring.at[b], accm.at[pl.ds(0, CGS)],
                                      asems[b]).wait()
                pltpu.make_async_copy(vbufs[b].at[pl.ds(0, 2 * CGS)],
                                      acc4.at[pl.ds(0, 2 * CGS)],
                                      asems[b]).wait()
                pltpu.make_async_copy(vbufs[b].at[pl.ds(0, 2 * CGS)],
                                      acc4.at[pl.ds(0, 2 * CGS)],
                                      asems[b]).wait()

            handles.append(pltpu.async_copy(m_h.at[pl.ds(off, CGS)],
                                            mring.at[b], asems[b]))
            handles.append(pltpu.async_copy(a_h.at[pl.ds(off * 4, 4 * CGS)],
                                            vbufs[b], asems[b]))
        for b in range(NBS):
            j = g * NBS + b
            handles[2 * b].wait()
            handles[2 * b + 1].wait()
            for g8 in range(4 * CGS // 16):
                lanes = lax.iota(jnp.int32, 16)
                e = j * CGS + g8 * 4 + lanes // 4
                rv = plsc.load_gather(idxf, [e])
                el = rv * 4 + lax.rem(lanes, 4)
                if g8 < 2 * CGS // 16:
                    plsc.store_scatter(eb0[b], [g8 * 16 + lanes], el)
                else:
                    plsc.store_scatter(eb1[b],
                                       [(g8 - 2 * CGS // 16) * 16 + lanes], el)
            pltpu.async_copy(mring.at[b], accm.at[idxg.at[pb, b]],
                             asems[b], add=True)
            pltpu.async_copy(vbufs[b].at[pl.ds(0, 2 * CGS)], acc4.at[eb0[b]],
                             asems[b], add=True)
            pltpu.async_copy(vbufs[b].at[pl.ds(2 * CGS, 2 * CGS)],
                             acc4.at[eb1[b]], asems[b], add=True)

        @pl.when(g + 1 < NGS)
        def _():
            pltpu.async_copy(r_t.at[g + 1], idxg.at[1 - pb], isem)

        return ()

    lax.fori_loop(0, NGS, grp, ())
    for b in range(NBS):
        pltpu.make_async_copy(mring.at[b], accm.at[pl.ds(0, CGS)],
                              asems[b]).wait()
        pltpu.make_async_copy(vbufs[b].at[pl.ds(0, 2 * CGS)],
                              acc4.at[pl.ds(0, 2 * CGS)], asems[b]).wait()
        pltpu.make_async_copy(vbufs[b].at[pl.ds(0, 2 * CGS)],
                              acc4.at[pl.ds(0, 2 * CGS)], asems[b]).wait()
    plsc.subcore_barrier()

    pltpu.sync_copy(acc4.at[pl.ds(sid * 4 * RPT, 4 * RPT)],
                    oa_h.at[cid].at[pl.ds(sid * 4 * RPT, 4 * RPT)])

    @pl.when(sid < NS - 1)
    def _():
        pltpu.sync_copy(accm.at[pl.ds(sid * RPT, RPT)],
                        om_h.at[cid].at[pl.ds(sid * RPT, RPT)])

    @pl.when(sid == NS - 1)
    def _():
        pltpu.sync_copy(accm.at[pl.ds((NS - 1) * RPT, 400)],
                        om_h.at[cid].at[pl.ds((NS - 1) * RPT, 400)])


def _sc_scatter_aux(m, aux_flat, row4d, row2, zer, zer1):
    f32 = jnp.float32
    fn = pl.kernel(
        _scatter_aux_body,
        out_type=[jax.ShapeDtypeStruct((2, N, H), f32),
                  jax.ShapeDtypeStruct((2, 4 * NP), f32)],
        mesh=plsc.VectorSubcoreMesh(core_axis_name="c", subcore_axis_name="s"),
        scratch_types=[pltpu.VMEM((2, NBS, CGS), jnp.int32),
                       pltpu.VMEM((E // 32,), jnp.int32),
                       pltpu.VMEM((NBS, CGS, H), f32),
                       pltpu.VMEM_SHARED((NP, H), f32),
                       pltpu.VMEM_SHARED((4 * NP,), f32),
                       pltpu.VMEM((4 * RPT,), f32)]
        + [pltpu.VMEM((4 * CGS,), f32)] * NBS
        + [pltpu.VMEM((2 * CGS,), jnp.int32)] * (2 * NBS)
        + [pltpu.SemaphoreType.DMA] * (NBS + 2),
        compiler_params=pltpu.CompilerParams(needs_layout_passes=False),
    )
    return fn(m, aux_flat, row4d, row2, zer, zer1)


NB3 = 5
CG3 = 80
CPT3 = (E // 32) // CG3


def _drr_body(x_h, r_h, c_h, o_h, xv, ridx, cidx, *rest):
    obufs, osems = rest[:NB3], rest[NB3:]
    cid = lax.axis_index("c")
    sid = lax.axis_index("s")
    tid = cid * NS + sid
    base = tid * (E // 32)
    pltpu.sync_copy(x_h, xv)
    pltpu.sync_copy(r_h.at[tid], ridx)
    pltpu.sync_copy(c_h.at[tid], cidx)
    lanes = lax.iota(jnp.int32, 16)

    def grp_fn(g, _):
        for b in range(NB3):
            j = g * NB3 + b

            @pl.when(g > 0)
            def _():
                pltpu.make_async_copy(obufs[b], o_h.at[pl.ds(0, 4 * CG3)],
                                      osems[b]).wait()

            for grp in range(CG3 // 16):
                ev = j * CG3 + grp * 16 + lanes
                rid = plsc.load_gather(ridx, [ev])
                cidv = plsc.load_gather(cidx, [ev])
                d0 = (plsc.load_gather(xv, [rid * 4])
                      - plsc.load_gather(xv, [cidv * 4]))
                d1 = (plsc.load_gather(xv, [rid * 4 + 1])
                      - plsc.load_gather(xv, [cidv * 4 + 1]))
                d2 = (plsc.load_gather(xv, [rid * 4 + 2])
                      - plsc.load_gather(xv, [cidv * 4 + 2]))
                r2 = d0 * d0 + d1 * d1 + d2 * d2
                pos = (grp * 16 + lanes) * 4
                plsc.store_scatter(obufs[b], [pos], d0)
                plsc.store_scatter(obufs[b], [pos + 1], d1)
                plsc.store_scatter(obufs[b], [pos + 2], d2)
                plsc.store_scatter(obufs[b], [pos + 3], r2)
            pltpu.async_copy(obufs[b], o_h.at[pl.ds((base + j * CG3) * 4,
                                                    4 * CG3)], osems[b])
        return ()

    lax.fori_loop(0, CPT3 // NB3, grp_fn, ())
    for b in range(NB3):
        pltpu.make_async_copy(obufs[b], o_h.at[pl.ds(0, 4 * CG3)],
                              osems[b]).wait()


def _sc_drr(x_flat, row2, col2):
    f32 = jnp.float32
    fn = pl.kernel(
        _drr_body,
        out_type=jax.ShapeDtypeStruct((E * 4,), f32),
        mesh=plsc.VectorSubcoreMesh(core_axis_name="c", subcore_axis_name="s"),
        scratch_types=[pltpu.VMEM((4 * N,), f32),
                       pltpu.VMEM((E // 32,), jnp.int32),
                       pltpu.VMEM((E // 32,), jnp.int32)]
        + [pltpu.VMEM((4 * CG3,), f32)] * NB3
        + [pltpu.SemaphoreType.DMA] * NB3,
        compiler_params=pltpu.CompilerParams(needs_layout_passes=False),
    )
    return fn(x_flat, row2, col2)


def kernel(atom_feature, coords, edge_index, edge_attr, params):
    f32 = jnp.float32
    row = edge_index[0].astype(jnp.int32)
    col = edge_index[1].astype(jnp.int32)
    x40 = jnp.pad(coords.astype(f32), ((0, 0), (0, 1)))

    w_in, b_in = params['emb_in']
    wo, bo = params['emb_out']
    lps = params['layers']

    def split_l(lp):
        w1, b1 = lp['edge1']
        return dict(
            w1a=w1[:H], w1b=w1[H:2 * H], rvec=w1[2 * H:2 * H + 1],
            w1e=w1[2 * H + 1:], b1=b1.reshape(1, H),
            w2=lp['edge2'][0], b2=lp['edge2'][1].reshape(1, H),
            cw1=lp['coord1'][0], cb1=lp['coord1'][1].reshape(1, H),
            c2w=lp['coord2_w'].reshape(1, H),
            nw1a=lp['node1'][0][:H], nw1b=lp['node1'][0][H:],
            nb1=lp['node1'][1].reshape(1, H),
            nw2=lp['node2'][0], nb2=lp['node2'][1].reshape(1, H),
        )

    l1, l2 = split_l(lps[0]), split_l(lps[1])
    row3d = row.reshape(NS, NGG, NBG, CGG)
    col3d = col.reshape(NS, NGG, NBG, CGG)
    row4d = row.reshape(32, NGS, NBS, CGS)
    row2 = row.reshape(32, E // 32)
    col2 = col.reshape(32, E // 32)
    zer = jnp.zeros((CGS, H), f32)
    zer1 = jnp.zeros((4 * RPT,), f32)

    h, ta, tb = _node_pre(atom_feature, w_in, b_in.reshape(1, H),
                          l1['w1a'], l1['w1b'])
    ga, gb = _sc_gather(ta, tb, row3d, col3d)
    drr = _sc_drr(x40.reshape(4 * N), row2, col2).reshape(E, 4)
    m, aux = _edge1(ga, gb, drr, edge_attr, l1['w1e'], l1['rvec'], l1['b1'],
                    l1['w2'], l1['b2'], l1['cw1'], l1['cb1'], l1['c2w'])
    mp, oa = _sc_scatter_aux(m, aux.reshape(4 * E), row4d, row2, zer, zer1)
    ap = oa.reshape(2, NP, 4)
    h2, ta2, tb2, x41 = _node1(h, mp, ap, x40, l1['nw1a'], l1['nw1b'],
                               l1['nb1'], l1['nw2'], l1['nb2'],
                               l2['w1a'], l2['w1b'])

    ga2, gb2 = _sc_gather(ta2, tb2, row3d, col3d)
    drr2 = _sc_drr(x41.reshape(4 * N), row2, col2).reshape(E, 4)
    m2 = _edge2(ga2, gb2, drr2, edge_attr, l2['w1e'], l2['rvec'], l2['b1'],
                l2['w2'], l2['b2'])
    mp2 = _sc_scatter(m2, row4d, zer)
    return _node_out(h2, mp2, l2['nw1a'], l2['nw1b'], l2['nb1'],
                     l2['nw2'], l2['nb2'], wo, bo.reshape(1, H))
